# baseline (speedup 1.0000x reference)
import functools

import jax
import jax.numpy as jnp
from jax import lax
from jax.experimental import pallas as pl
from jax.experimental.pallas import tpu as pltpu
from jax.experimental.pallas import tpu_sc as plsc

_B = 8
_N = 4096
_S = 1024
_K = 32
_R2 = 0.1 * 0.1
_CW = 80
_M = _B * _S * _K
_TM = 512
_L = 16

def _fps_body(x_hbm, y_hbm, z_hbm, ocx, ocy, ocz, xv, yv, zv, dv, cxv, cyv, czv):
    w = lax.axis_index("s") * 2 + lax.axis_index("c")

    @pl.when(w < _B)
    def _():
        pltpu.sync_copy(x_hbm.at[w], xv)
        pltpu.sync_copy(y_hbm.at[w], yv)
        pltpu.sync_copy(z_hbm.at[w], zv)
        lanes = lax.iota(jnp.int32, _L)
        lane0 = lanes == 0
        big = jnp.full((_L,), 1e10, jnp.float32)

        def initc(i, carry):
            dv[pl.ds(i * _L, _L)] = big
            return carry

        lax.fori_loop(0, _N // _L, initc, 0)

        zero16 = jnp.zeros((_L,), jnp.int32)

        def step(i, best0):
            bvec0 = jnp.full((_L,), best0, jnp.int32)
            cx = plsc.load_gather(xv, [bvec0])
            cy = plsc.load_gather(yv, [bvec0])
            cz = plsc.load_gather(zv, [bvec0])
            ivec = jnp.full((_L,), i, jnp.int32)
            plsc.store_scatter(cxv, [ivec], cx, mask=lane0)
            plsc.store_scatter(cyv, [ivec], cy, mask=lane0)
            plsc.store_scatter(czv, [ivec], cz, mask=lane0)

            def chunk(cc, car):
                mx, bi = car
                for k in range(4):
                    sl = pl.ds(cc * (4 * _L) + k * _L, _L)
                    dx = xv[sl] - cx
                    dy = yv[sl] - cy
                    dz = zv[sl] - cz
                    d = dx * dx + dy * dy
                    d = d + dz * dz
                    dn = jnp.minimum(dv[sl], d)
                    dv[sl] = dn
                    gt = dn > mx
                    mx = jnp.where(gt, dn, mx)
                    bi = jnp.where(gt, cc * (4 * _L) + k * _L + lanes, bi)
                return mx, bi

            mx, bi = lax.fori_loop(
                0, _N // (4 * _L), chunk,
                (jnp.full((_L,), -1.0, jnp.float32), zero16))
            m = jnp.max(mx)
            cand = jnp.where(mx == m, bi, _N)
            return jnp.min(cand)

        lax.fori_loop(0, _S, step, jnp.int32(0))
        pltpu.sync_copy(cxv, ocx.at[w])
        pltpu.sync_copy(cyv, ocy.at[w])
        pltpu.sync_copy(czv, ocz.at[w])


def _ball_body(x_hbm, y_hbm, z_hbm, cx_hbm, cy_hbm, cz_hbm, r_hbm, x0_hbm,
               xv, yv, zv, ccx, ccy, ccz, idxv, rows, sem):
    w = lax.axis_index("s") * 2 + lax.axis_index("c")
    b = w // 4
    q = w % 4
    sc = _S // 4
    pltpu.sync_copy(x_hbm.at[b], xv)
    pltpu.sync_copy(y_hbm.at[b], yv)
    pltpu.sync_copy(z_hbm.at[b], zv)
    pltpu.sync_copy(cx_hbm.at[b, pl.ds(q * sc, sc)], ccx)
    pltpu.sync_copy(cy_hbm.at[b, pl.ds(q * sc, sc)], ccy)
    pltpu.sync_copy(cz_hbm.at[b, pl.ds(q * sc, sc)], ccz)
    lanes = lax.iota(jnp.int32, _L)
    gb = b * _N

    def per_center(si, carry):
        sv = jnp.full((_L,), si, jnp.int32)
        ctx = plsc.load_gather(ccx, [sv])
        cty = plsc.load_gather(ccy, [sv])
        ctz = plsc.load_gather(ccz, [sv])

        def cond(st):
            ci, found, first = st
            return jnp.logical_and(ci < _N // _L, found < _K)

        def body(st):
            ci, found, first = st
            sl = pl.ds(ci * _L, _L)
            dx = xv[sl] - ctx
            dy = yv[sl] - cty
            dz = zv[sl] - ctz
            d = dx * dx + dy * dy
            d = d + dz * dz
            msk = d < _R2
            anyh = jnp.any(msk)

            def hitfn(args):
                found, first = args
                csum = jnp.cumsum(msk.astype(jnp.int32))
                rank = found + csum - 1
                okm = jnp.logical_and(msk, rank < _K)
                gidx = gb + ci * _L + lanes
                plsc.store_scatter(idxv, [rank], gidx, mask=okm)
                first = jnp.where(
                    found == 0,
                    jnp.min(jnp.where(msk, gidx, jnp.int32(2**30))),
                    first)
                return found + jnp.max(csum), first

            def nohit(args):
                return args

            found, first = lax.cond(anyh, hitfn, nohit, (found, first))
            return ci + 1, found, first

        _, found, first = lax.while_loop(
            cond, body, (jnp.int32(0), jnp.int32(0), gb))
        found = jnp.minimum(found, _K)
        fvec = jnp.full((_L,), first, jnp.int32)
        for j in range(_K // _L):
            sl = pl.ds(j * _L, _L)
            keep = (j * _L + lanes) < found
            idxv[sl] = jnp.where(keep, idxv[sl], fvec)

        pltpu.async_copy(r_hbm.at[idxv], rows, sem).wait()

        for coord, ctv in ((64, ctx), (65, cty), (66, ctz)):
            colv = jnp.full((_L,), coord, jnp.int32)
            for j in range(_K // _L):
                ridx = j * _L + lanes
                vals = plsc.load_gather(rows, [ridx, colv])
                plsc.store_scatter(rows, [ridx, colv], vals - ctv)

        gc = ((b * _S + q * sc) + si) * _K
        pltpu.sync_copy(rows, x0_hbm.at[pl.ds(gc, _K)])
        return carry

    lax.fori_loop(0, sc, per_center, 0)


@functools.lru_cache(maxsize=None)
def _sc_kernels():
    mesh = plsc.VectorSubcoreMesh(core_axis_name="c", subcore_axis_name="s",
                                  num_cores=2, num_subcores=16)
    cparams = pltpu.CompilerParams(needs_layout_passes=False,
                                   use_tc_tiling_on_sc=False)
    fps = pl.kernel(
        _fps_body,
        out_type=[jax.ShapeDtypeStruct((_B, _S), jnp.float32)] * 3,
        mesh=mesh,
        compiler_params=cparams,
        scratch_types=[
            pltpu.VMEM((_N,), jnp.float32),
            pltpu.VMEM((_N,), jnp.float32),
            pltpu.VMEM((_N,), jnp.float32),
            pltpu.VMEM((_N,), jnp.float32),
            pltpu.VMEM((_S,), jnp.float32),
            pltpu.VMEM((_S,), jnp.float32),
            pltpu.VMEM((_S,), jnp.float32),
        ],
    )
    ball = pl.kernel(
        _ball_body,
        out_type=jax.ShapeDtypeStruct((_M, _CW), jnp.float32),
        mesh=mesh,
        compiler_params=cparams,
        scratch_types=[
            pltpu.VMEM((_N,), jnp.float32),
            pltpu.VMEM((_N,), jnp.float32),
            pltpu.VMEM((_N,), jnp.float32),
            pltpu.VMEM((_S // 4,), jnp.float32),
            pltpu.VMEM((_S // 4,), jnp.float32),
            pltpu.VMEM((_S // 4,), jnp.float32),
            pltpu.VMEM((_K,), jnp.int32),
            pltpu.VMEM((_K, _CW), jnp.float32),
            pltpu.SemaphoreType.DMA,
        ],
    )
    return fps, ball


def _moments_body(x_ref, g_ref, s_ref):
    i = pl.program_id(0)

    @pl.when(i == 0)
    def _():
        g_ref[...] = jnp.zeros_like(g_ref)
        s_ref[...] = jnp.zeros_like(s_ref)

    x = x_ref[...]
    g_ref[...] += lax.dot_general(x, x, (((0,), (0,)), ((), ())),
                                  preferred_element_type=jnp.float32)
    s_ref[...] += jnp.sum(x, axis=0, keepdims=True)


def _moments(x):
    m, c = x.shape
    return pl.pallas_call(
        _moments_body,
        grid=(m // _TM,),
        in_specs=[pl.BlockSpec((_TM, c), lambda i: (i, 0))],
        out_specs=[pl.BlockSpec((c, c), lambda i: (0, 0)),
                   pl.BlockSpec((1, c), lambda i: (0, 0))],
        out_shape=[jax.ShapeDtypeStruct((c, c), jnp.float32),
                   jax.ShapeDtypeStruct((1, c), jnp.float32)],
    )(x)


def _layer_body(x_ref, w_ref, b_ref, y_ref, g_ref, s_ref):
    i = pl.program_id(0)

    @pl.when(i == 0)
    def _():
        g_ref[...] = jnp.zeros_like(g_ref)
        s_ref[...] = jnp.zeros_like(s_ref)

    y = lax.dot_general(x_ref[...], w_ref[...], (((1,), (0,)), ((), ())),
                        preferred_element_type=jnp.float32)
    y = jnp.maximum(y + b_ref[...], 0.0)
    y_ref[...] = y
    g_ref[...] += lax.dot_general(y, y, (((0,), (0,)), ((), ())),
                                  preferred_element_type=jnp.float32)
    s_ref[...] += jnp.sum(y, axis=0, keepdims=True)


def _layer(x, w, b):
    m, c = x.shape
    o = w.shape[1]
    return pl.pallas_call(
        _layer_body,
        grid=(m // _TM,),
        in_specs=[pl.BlockSpec((_TM, c), lambda i: (i, 0)),
                  pl.BlockSpec((c, o), lambda i: (0, 0)),
                  pl.BlockSpec((1, o), lambda i: (0, 0))],
        out_specs=[pl.BlockSpec((_TM, o), lambda i: (i, 0)),
                   pl.BlockSpec((o, o), lambda i: (0, 0)),
                   pl.BlockSpec((1, o), lambda i: (0, 0))],
        out_shape=[jax.ShapeDtypeStruct((m, o), jnp.float32),
                   jax.ShapeDtypeStruct((o, o), jnp.float32),
                   jax.ShapeDtypeStruct((1, o), jnp.float32)],
    )(x, w, b.reshape(1, o))


def _final_body(x_ref, w_ref, b_ref, o_ref):
    y = lax.dot_general(x_ref[...], w_ref[...], (((1,), (0,)), ((), ())),
                        preferred_element_type=jnp.float32)
    y = jnp.maximum(y + b_ref[...], 0.0)
    parts = [jnp.max(y[j * _K:(j + 1) * _K, :], axis=0, keepdims=True)
             for j in range(_TM // _K)]
    o_ref[...] = jnp.concatenate(parts, axis=0)


def _final(x, w, b):
    m, c = x.shape
    o = w.shape[1]
    return pl.pallas_call(
        _final_body,
        grid=(m // _TM,),
        in_specs=[pl.BlockSpec((_TM, c), lambda i: (i, 0)),
                  pl.BlockSpec((c, o), lambda i: (0, 0)),
                  pl.BlockSpec((1, o), lambda i: (0, 0))],
        out_specs=pl.BlockSpec((_TM // _K, o), lambda i: (i, 0)),
        out_shape=jax.ShapeDtypeStruct((m // _K, o), jnp.float32),
    )(x, w, b.reshape(1, o))


def _fold(G, S, W, b, g, be):
    mu = S[0] / _M
    muW = mu @ W
    mean_y = muW + b
    T = G @ W
    ey2 = jnp.sum(W * T, axis=0) / _M + 2.0 * b * muW + b * b
    var = ey2 - mean_y * mean_y
    scale = g / jnp.sqrt(var + 1e-5)
    return W * scale[None, :], (b - mean_y) * scale + be


def kernel(xyz, points, W0, b0, g0, be0, W1, b1, g1, be1, W2, b2, g2, be2):
    xb = xyz[:, 0, :]
    yb = xyz[:, 1, :]
    zb = xyz[:, 2, :]
    fps_call, ball_call = _sc_kernels()
    cx, cy, cz = fps_call(xb, yb, zb)
    new_xyz = jnp.stack([cx, cy, cz], axis=1)

    R = jnp.concatenate(
        [points.transpose(0, 2, 1), xyz.transpose(0, 2, 1),
         jnp.zeros((_B, _N, _CW - 67), jnp.float32)], axis=-1,
    ).reshape(_B * _N, _CW)
    x0 = ball_call(xb, yb, zb, cx, cy, cz, R)

    W0e = jnp.zeros((_CW, W0.shape[0]), jnp.float32)
    W0e = W0e.at[0:64, :].set(W0[:, 3:67].T)
    W0e = W0e.at[64:67, :].set(W0[:, 0:3].T)

    G0, S0 = _moments(x0)
    W0f, b0f = _fold(G0, S0, W0e, b0, g0, be0)
    x1, G1, S1 = _layer(x0, W0f, b0f)
    W1f, b1f = _fold(G1, S1, W1.T, b1, g1, be1)
    x2, G2, S2 = _layer(x1, W1f, b1f)
    W2f, b2f = _fold(G2, S2, W2.T, b2, g2, be2)
    feats = _final(x2, W2f, b2f)
    new_features = feats.reshape(_B, _S, W2.shape[0]).transpose(0, 2, 1)
    return new_xyz, new_features


# speedup vs baseline: 1.6782x; 1.6782x over previous
import functools

import jax
import jax.numpy as jnp
from jax import lax
from jax.experimental import pallas as pl
from jax.experimental.pallas import tpu as pltpu
from jax.experimental.pallas import tpu_sc as plsc

_B = 8
_N = 4096
_S = 1024
_K = 32
_R2 = 0.1 * 0.1
_CW = 80
_M = _B * _S * _K
_TM = 512
_L = 16

def _fps_body(x_hbm, y_hbm, z_hbm, ocx, ocy, ocz, xv, yv, zv, dv, cxv, cyv, czv):
    w = lax.axis_index("s") * 2 + lax.axis_index("c")

    @pl.when(w < _B)
    def _():
        pltpu.sync_copy(x_hbm.at[w], xv)
        pltpu.sync_copy(y_hbm.at[w], yv)
        pltpu.sync_copy(z_hbm.at[w], zv)
        lanes = lax.iota(jnp.int32, _L)
        lane0 = lanes == 0
        big = jnp.full((_L,), 1e10, jnp.float32)

        def initc(i, carry):
            dv[pl.ds(i * _L, _L)] = big
            return carry

        lax.fori_loop(0, _N // _L, initc, 0)

        zero16 = jnp.zeros((_L,), jnp.int32)

        def step(i, best0):
            bvec0 = jnp.full((_L,), best0, jnp.int32)
            cx = plsc.load_gather(xv, [bvec0])
            cy = plsc.load_gather(yv, [bvec0])
            cz = plsc.load_gather(zv, [bvec0])
            ivec = jnp.full((_L,), i, jnp.int32)
            plsc.store_scatter(cxv, [ivec], cx, mask=lane0)
            plsc.store_scatter(cyv, [ivec], cy, mask=lane0)
            plsc.store_scatter(czv, [ivec], cz, mask=lane0)

            def chunk(cc, car):
                mx, bi = car
                for k in range(4):
                    sl = pl.ds(cc * (4 * _L) + k * _L, _L)
                    dx = xv[sl] - cx
                    dy = yv[sl] - cy
                    dz = zv[sl] - cz
                    d = dx * dx + dy * dy
                    d = d + dz * dz
                    dn = jnp.minimum(dv[sl], d)
                    dv[sl] = dn
                    gt = dn > mx
                    mx = jnp.where(gt, dn, mx)
                    bi = jnp.where(gt, cc * (4 * _L) + k * _L + lanes, bi)
                return mx, bi

            mx, bi = lax.fori_loop(
                0, _N // (4 * _L), chunk,
                (jnp.full((_L,), -1.0, jnp.float32), zero16))
            m = jnp.max(mx)
            cand = jnp.where(mx == m, bi, _N)
            return jnp.min(cand)

        lax.fori_loop(0, _S, step, jnp.int32(0))
        pltpu.sync_copy(cxv, ocx.at[w])
        pltpu.sync_copy(cyv, ocy.at[w])
        pltpu.sync_copy(czv, ocz.at[w])


def _ball_body(x_hbm, y_hbm, z_hbm, cx_hbm, cy_hbm, cz_hbm, r_hbm, x0_hbm,
               xv, yv, zv, ccx, ccy, ccz, cntb, idxb, rows, sem):
    w = lax.axis_index("s") * 2 + lax.axis_index("c")
    b = w // 4
    q = w % 4
    sc = _S // 4
    pltpu.sync_copy(x_hbm.at[b], xv)
    pltpu.sync_copy(y_hbm.at[b], yv)
    pltpu.sync_copy(z_hbm.at[b], zv)
    pltpu.sync_copy(cx_hbm.at[b, pl.ds(q * sc, sc)], ccx)
    pltpu.sync_copy(cy_hbm.at[b, pl.ds(q * sc, sc)], ccy)
    pltpu.sync_copy(cz_hbm.at[b, pl.ds(q * sc, sc)], ccz)
    lanes = lax.iota(jnp.int32, _L)
    gb = b * _N
    gbv = jnp.full((_L,), gb, jnp.int32)

    def group(g, carry):
        slc = pl.ds(g * _L, _L)
        cxg = ccx[slc]
        cyg = ccy[slc]
        czg = ccz[slc]
        slotbase = (g * _L + lanes) * _K
        plsc.store_scatter(idxb, [slotbase], gbv)

        def pchunk(c, cnt):
            sl = pl.ds(c * _L, _L)
            xs = xv[sl]
            ys = yv[sl]
            zs = zv[sl]
            for k in range(_L):
                dx = cxg - xs[k]
                dy = cyg - ys[k]
                dz = czg - zs[k]
                d = dx * dx + dy * dy
                d = d + dz * dz
                msk = d < _R2
                okm = jnp.logical_and(msk, cnt < _K)
                gidx = jnp.full((_L,), gb + c * _L + k, jnp.int32)
                plsc.store_scatter(idxb, [slotbase + cnt], gidx, mask=okm)
                cnt = cnt + msk.astype(jnp.int32)
            return cnt

        cnt = lax.fori_loop(0, _N // _L, pchunk, jnp.zeros((_L,), jnp.int32))
        cntb[slc] = jnp.minimum(cnt, _K)
        return carry

    lax.fori_loop(0, sc // _L, group, 0)

    def block(blk, carry):
        for t in range(4):
            cl = blk * 4 + t
            clv = jnp.full((_L,), cl, jnp.int32)
            cntv = plsc.load_gather(cntb, [clv])
            firstv = plsc.load_gather(idxb, [clv * _K])
            for j in range(_K // _L):
                sl = pl.ds(cl * _K + j * _L, _L)
                keep = (j * _L + lanes) < cntv
                idxb[sl] = jnp.where(keep, idxb[sl], firstv)
        pltpu.async_copy(r_hbm.at[idxb.at[pl.ds(blk * 128, 128)]],
                         rows, sem).wait()
        for t in range(4):
            cl = blk * 4 + t
            clv = jnp.full((_L,), cl, jnp.int32)
            ctx = plsc.load_gather(ccx, [clv])
            cty = plsc.load_gather(ccy, [clv])
            ctz = plsc.load_gather(ccz, [clv])
            for coord, ctv in ((64, ctx), (65, cty), (66, ctz)):
                colv = jnp.full((_L,), coord, jnp.int32)
                for j in range(_K // _L):
                    ridx = t * _K + j * _L + lanes
                    vals = plsc.load_gather(rows, [ridx, colv])
                    plsc.store_scatter(rows, [ridx, colv], vals - ctv)
        gc = (b * _S + q * sc + blk * 4) * _K
        pltpu.sync_copy(rows, x0_hbm.at[pl.ds(gc, 128)])
        return carry

    lax.fori_loop(0, sc // 4, block, 0)


@functools.lru_cache(maxsize=None)
def _sc_kernels():
    mesh = plsc.VectorSubcoreMesh(core_axis_name="c", subcore_axis_name="s",
                                  num_cores=2, num_subcores=16)
    cparams = pltpu.CompilerParams(needs_layout_passes=False,
                                   use_tc_tiling_on_sc=False)
    fps = pl.kernel(
        _fps_body,
        out_type=[jax.ShapeDtypeStruct((_B, _S), jnp.float32)] * 3,
        mesh=mesh,
        compiler_params=cparams,
        scratch_types=[
            pltpu.VMEM((_N,), jnp.float32),
            pltpu.VMEM((_N,), jnp.float32),
            pltpu.VMEM((_N,), jnp.float32),
            pltpu.VMEM((_N,), jnp.float32),
            pltpu.VMEM((_S,), jnp.float32),
            pltpu.VMEM((_S,), jnp.float32),
            pltpu.VMEM((_S,), jnp.float32),
        ],
    )
    ball = pl.kernel(
        _ball_body,
        out_type=jax.ShapeDtypeStruct((_M, _CW), jnp.float32),
        mesh=mesh,
        compiler_params=cparams,
        scratch_types=[
            pltpu.VMEM((_N,), jnp.float32),
            pltpu.VMEM((_N,), jnp.float32),
            pltpu.VMEM((_N,), jnp.float32),
            pltpu.VMEM((_S // 4,), jnp.float32),
            pltpu.VMEM((_S // 4,), jnp.float32),
            pltpu.VMEM((_S // 4,), jnp.float32),
            pltpu.VMEM((_S // 4,), jnp.int32),
            pltpu.VMEM((_S // 4 * _K,), jnp.int32),
            pltpu.VMEM((128, _CW), jnp.float32),
            pltpu.SemaphoreType.DMA,
        ],
    )
    return fps, ball


def _moments_body(x_ref, g_ref, s_ref):
    i = pl.program_id(0)

    @pl.when(i == 0)
    def _():
        g_ref[...] = jnp.zeros_like(g_ref)
        s_ref[...] = jnp.zeros_like(s_ref)

    x = x_ref[...]
    g_ref[...] += lax.dot_general(x, x, (((0,), (0,)), ((), ())),
                                  preferred_element_type=jnp.float32)
    s_ref[...] += jnp.sum(x, axis=0, keepdims=True)


def _moments(x):
    m, c = x.shape
    return pl.pallas_call(
        _moments_body,
        grid=(m // _TM,),
        in_specs=[pl.BlockSpec((_TM, c), lambda i: (i, 0))],
        out_specs=[pl.BlockSpec((c, c), lambda i: (0, 0)),
                   pl.BlockSpec((1, c), lambda i: (0, 0))],
        out_shape=[jax.ShapeDtypeStruct((c, c), jnp.float32),
                   jax.ShapeDtypeStruct((1, c), jnp.float32)],
    )(x)


def _layer_body(x_ref, w_ref, b_ref, y_ref, g_ref, s_ref):
    i = pl.program_id(0)

    @pl.when(i == 0)
    def _():
        g_ref[...] = jnp.zeros_like(g_ref)
        s_ref[...] = jnp.zeros_like(s_ref)

    y = lax.dot_general(x_ref[...], w_ref[...], (((1,), (0,)), ((), ())),
                        preferred_element_type=jnp.float32)
    y = jnp.maximum(y + b_ref[...], 0.0)
    y_ref[...] = y
    g_ref[...] += lax.dot_general(y, y, (((0,), (0,)), ((), ())),
                                  preferred_element_type=jnp.float32)
    s_ref[...] += jnp.sum(y, axis=0, keepdims=True)


def _layer(x, w, b):
    m, c = x.shape
    o = w.shape[1]
    return pl.pallas_call(
        _layer_body,
        grid=(m // _TM,),
        in_specs=[pl.BlockSpec((_TM, c), lambda i: (i, 0)),
                  pl.BlockSpec((c, o), lambda i: (0, 0)),
                  pl.BlockSpec((1, o), lambda i: (0, 0))],
        out_specs=[pl.BlockSpec((_TM, o), lambda i: (i, 0)),
                   pl.BlockSpec((o, o), lambda i: (0, 0)),
                   pl.BlockSpec((1, o), lambda i: (0, 0))],
        out_shape=[jax.ShapeDtypeStruct((m, o), jnp.float32),
                   jax.ShapeDtypeStruct((o, o), jnp.float32),
                   jax.ShapeDtypeStruct((1, o), jnp.float32)],
    )(x, w, b.reshape(1, o))


def _final_body(x_ref, w_ref, b_ref, o_ref):
    y = lax.dot_general(x_ref[...], w_ref[...], (((1,), (0,)), ((), ())),
                        preferred_element_type=jnp.float32)
    y = jnp.maximum(y + b_ref[...], 0.0)
    parts = [jnp.max(y[j * _K:(j + 1) * _K, :], axis=0, keepdims=True)
             for j in range(_TM // _K)]
    o_ref[...] = jnp.concatenate(parts, axis=0)


def _final(x, w, b):
    m, c = x.shape
    o = w.shape[1]
    return pl.pallas_call(
        _final_body,
        grid=(m // _TM,),
        in_specs=[pl.BlockSpec((_TM, c), lambda i: (i, 0)),
                  pl.BlockSpec((c, o), lambda i: (0, 0)),
                  pl.BlockSpec((1, o), lambda i: (0, 0))],
        out_specs=pl.BlockSpec((_TM // _K, o), lambda i: (i, 0)),
        out_shape=jax.ShapeDtypeStruct((m // _K, o), jnp.float32),
    )(x, w, b.reshape(1, o))


def _fold(G, S, W, b, g, be):
    mu = S[0] / _M
    muW = mu @ W
    mean_y = muW + b
    T = G @ W
    ey2 = jnp.sum(W * T, axis=0) / _M + 2.0 * b * muW + b * b
    var = ey2 - mean_y * mean_y
    scale = g / jnp.sqrt(var + 1e-5)
    return W * scale[None, :], (b - mean_y) * scale + be


def kernel(xyz, points, W0, b0, g0, be0, W1, b1, g1, be1, W2, b2, g2, be2):
    xb = xyz[:, 0, :]
    yb = xyz[:, 1, :]
    zb = xyz[:, 2, :]
    fps_call, ball_call = _sc_kernels()
    cx, cy, cz = fps_call(xb, yb, zb)
    new_xyz = jnp.stack([cx, cy, cz], axis=1)

    R = jnp.concatenate(
        [points.transpose(0, 2, 1), xyz.transpose(0, 2, 1),
         jnp.zeros((_B, _N, _CW - 67), jnp.float32)], axis=-1,
    ).reshape(_B * _N, _CW)
    x0 = ball_call(xb, yb, zb, cx, cy, cz, R)

    W0e = jnp.zeros((_CW, W0.shape[0]), jnp.float32)
    W0e = W0e.at[0:64, :].set(W0[:, 3:67].T)
    W0e = W0e.at[64:67, :].set(W0[:, 0:3].T)

    G0, S0 = _moments(x0)
    W0f, b0f = _fold(G0, S0, W0e, b0, g0, be0)
    x1, G1, S1 = _layer(x0, W0f, b0f)
    W1f, b1f = _fold(G1, S1, W1.T, b1, g1, be1)
    x2, G2, S2 = _layer(x1, W1f, b1f)
    W2f, b2f = _fold(G2, S2, W2.T, b2, g2, be2)
    feats = _final(x2, W2f, b2f)
    new_features = feats.reshape(_B, _S, W2.shape[0]).transpose(0, 2, 1)
    return new_xyz, new_features


# speedup vs baseline: 2.6036x; 1.5515x over previous
import functools

import jax
import jax.numpy as jnp
from jax import lax
from jax.experimental import pallas as pl
from jax.experimental.pallas import tpu as pltpu
from jax.experimental.pallas import tpu_sc as plsc

_B = 8
_N = 4096
_S = 1024
_K = 32
_R2 = 0.1 * 0.1
_CW = 80
_M = _B * _S * _K
_TM = 512
_L = 16

def _fps_body(x_hbm, y_hbm, z_hbm, ocx, ocy, ocz,
              xv, yv, zv, dv, cxv, cyv, czv, pubv, exch, shr):
    c = lax.axis_index("c")
    s = lax.axis_index("s")
    b = c * 4 + s // 4
    q = s % 4
    nq = _N // 4
    qoff = q * nq
    sg = (s // 4) * 4
    lanes = lax.iota(jnp.int32, _L)
    lane0 = lanes == 0
    rowm = lanes & 3
    pltpu.sync_copy(x_hbm.at[b, pl.ds(qoff, nq)], xv)
    pltpu.sync_copy(y_hbm.at[b, pl.ds(qoff, nq)], yv)
    pltpu.sync_copy(z_hbm.at[b, pl.ds(qoff, nq)], zv)
    big = jnp.full((_L,), 1e10, jnp.float32)

    def initc(i, carry):
        dv[pl.ds(i * _L, _L)] = big
        return carry

    lax.fori_loop(0, nq // _L, initc, 0)

    def exchange(par, val, idxf, xc, yc, zc):
        p = jnp.broadcast_to(val, (_L,)).astype(jnp.float32)
        p = jnp.where(lanes == 1, idxf, p)
        p = jnp.where(lanes == 2, xc, p)
        p = jnp.where(lanes == 3, yc, p)
        p = jnp.where(lanes == 4, zc, p)
        pubv[...] = p
        pltpu.sync_copy(pubv, shr.at[par, s])
        plsc.subcore_barrier()
        pltpu.sync_copy(shr.at[par, pl.ds(sg, 4)], exch)
        vals = plsc.load_gather(exch, [rowm, jnp.full((_L,), 0, jnp.int32)])
        m = jnp.max(vals)
        idp = plsc.load_gather(exch, [rowm, jnp.full((_L,), 1, jnp.int32)])
        cand = jnp.where(vals == m, idp, jnp.float32(1e30))
        bidf = jnp.min(cand)
        wm = jnp.logical_and(vals == m, idp == bidf)
        xs = plsc.load_gather(exch, [rowm, jnp.full((_L,), 2, jnp.int32)])
        ys = plsc.load_gather(exch, [rowm, jnp.full((_L,), 3, jnp.int32)])
        zs = plsc.load_gather(exch, [rowm, jnp.full((_L,), 4, jnp.int32)])
        neg = jnp.float32(-1e30)
        nx = jnp.max(jnp.where(wm, xs, neg))
        ny = jnp.max(jnp.where(wm, ys, neg))
        nz = jnp.max(jnp.where(wm, zs, neg))
        return nx, ny, nz

    v0 = jnp.where(q == 0, jnp.float32(1.0), jnp.float32(0.0))
    x00 = xv[pl.ds(0, _L)][0]
    y00 = yv[pl.ds(0, _L)][0]
    z00 = zv[pl.ds(0, _L)][0]
    cx0, cy0, cz0 = exchange(1, v0, jnp.float32(0.0), x00, y00, z00)

    def step(i, carry):
        cx, cy, cz = carry

        @pl.when(q == 0)
        def _():
            ivec = jnp.full((_L,), i, jnp.int32)
            plsc.store_scatter(cxv, [ivec], jnp.full((_L,), cx, jnp.float32),
                               mask=lane0)
            plsc.store_scatter(cyv, [ivec], jnp.full((_L,), cy, jnp.float32),
                               mask=lane0)
            plsc.store_scatter(czv, [ivec], jnp.full((_L,), cz, jnp.float32),
                               mask=lane0)

        def chunk(cc, car):
            mx, bi = car
            for k in range(4):
                off = cc * (4 * _L) + k * _L
                sl = pl.ds(off, _L)
                dx = xv[sl] - cx
                dy = yv[sl] - cy
                dz = zv[sl] - cz
                d = dx * dx + dy * dy
                d = d + dz * dz
                dn = jnp.minimum(dv[sl], d)
                dv[sl] = dn
                gt = dn > mx
                mx = jnp.where(gt, dn, mx)
                bi = jnp.where(gt, off + lanes, bi)
            return mx, bi

        mx, bi = lax.fori_loop(
            0, nq // (4 * _L), chunk,
            (jnp.full((_L,), -1.0, jnp.float32), jnp.zeros((_L,), jnp.int32)))
        m = jnp.max(mx)
        cand = jnp.where(mx == m, bi, nq)
        bl = jnp.min(cand)
        cvx = plsc.load_gather(xv, [jnp.full((_L,), bl, jnp.int32)])
        cvy = plsc.load_gather(yv, [jnp.full((_L,), bl, jnp.int32)])
        cvz = plsc.load_gather(zv, [jnp.full((_L,), bl, jnp.int32)])
        bgf = (qoff + bl).astype(jnp.float32)
        return exchange(i & 1, m, bgf, cvx, cvy, cvz)

    lax.fori_loop(0, _S, step, (cx0, cy0, cz0))

    @pl.when(q == 0)
    def _():
        pltpu.sync_copy(cxv, ocx.at[b])
        pltpu.sync_copy(cyv, ocy.at[b])
        pltpu.sync_copy(czv, ocz.at[b])


def _ball_body(x_hbm, y_hbm, z_hbm, cx_hbm, cy_hbm, cz_hbm, r_hbm, x0_hbm,
               xv, yv, zv, ccx, ccy, ccz, cntb, idxb, rows, sem):
    w = lax.axis_index("s") * 2 + lax.axis_index("c")
    b = w // 4
    q = w % 4
    sc = _S // 4
    pltpu.sync_copy(x_hbm.at[b], xv)
    pltpu.sync_copy(y_hbm.at[b], yv)
    pltpu.sync_copy(z_hbm.at[b], zv)
    pltpu.sync_copy(cx_hbm.at[b, pl.ds(q * sc, sc)], ccx)
    pltpu.sync_copy(cy_hbm.at[b, pl.ds(q * sc, sc)], ccy)
    pltpu.sync_copy(cz_hbm.at[b, pl.ds(q * sc, sc)], ccz)
    lanes = lax.iota(jnp.int32, _L)
    gb = b * _N
    gbv = jnp.full((_L,), gb, jnp.int32)

    def group(g, carry):
        slc = pl.ds(g * _L, _L)
        cxg = ccx[slc]
        cyg = ccy[slc]
        czg = ccz[slc]
        slotbase = (g * _L + lanes) * _K
        plsc.store_scatter(idxb, [slotbase], gbv)

        def pchunk(c, cnt):
            sl = pl.ds(c * _L, _L)
            xs = xv[sl]
            ys = yv[sl]
            zs = zv[sl]
            for k in range(_L):
                dx = cxg - xs[k]
                dy = cyg - ys[k]
                dz = czg - zs[k]
                d = dx * dx + dy * dy
                d = d + dz * dz
                msk = d < _R2
                okm = jnp.logical_and(msk, cnt < _K)
                gidx = jnp.full((_L,), gb + c * _L + k, jnp.int32)
                plsc.store_scatter(idxb, [slotbase + cnt], gidx, mask=okm)
                cnt = cnt + msk.astype(jnp.int32)
            return cnt

        cnt = lax.fori_loop(0, _N // _L, pchunk, jnp.zeros((_L,), jnp.int32))
        cntb[slc] = jnp.minimum(cnt, _K)
        return carry

    lax.fori_loop(0, sc // _L, group, 0)

    def block(blk, carry):
        for t in range(4):
            cl = blk * 4 + t
            clv = jnp.full((_L,), cl, jnp.int32)
            cntv = plsc.load_gather(cntb, [clv])
            firstv = plsc.load_gather(idxb, [clv * _K])
            for j in range(_K // _L):
                sl = pl.ds(cl * _K + j * _L, _L)
                keep = (j * _L + lanes) < cntv
                idxb[sl] = jnp.where(keep, idxb[sl], firstv)
        pltpu.async_copy(r_hbm.at[idxb.at[pl.ds(blk * 128, 128)]],
                         rows, sem).wait()
        for t in range(4):
            cl = blk * 4 + t
            clv = jnp.full((_L,), cl, jnp.int32)
            ctx = plsc.load_gather(ccx, [clv])
            cty = plsc.load_gather(ccy, [clv])
            ctz = plsc.load_gather(ccz, [clv])
            for coord, ctv in ((64, ctx), (65, cty), (66, ctz)):
                colv = jnp.full((_L,), coord, jnp.int32)
                for j in range(_K // _L):
                    ridx = t * _K + j * _L + lanes
                    vals = plsc.load_gather(rows, [ridx, colv])
                    plsc.store_scatter(rows, [ridx, colv], vals - ctv)
        gc = (b * _S + q * sc + blk * 4) * _K
        pltpu.sync_copy(rows, x0_hbm.at[pl.ds(gc, 128)])
        return carry

    lax.fori_loop(0, sc // 4, block, 0)


@functools.lru_cache(maxsize=None)
def _sc_kernels():
    mesh = plsc.VectorSubcoreMesh(core_axis_name="c", subcore_axis_name="s",
                                  num_cores=2, num_subcores=16)
    cparams = pltpu.CompilerParams(needs_layout_passes=False,
                                   use_tc_tiling_on_sc=False)
    fps = pl.kernel(
        _fps_body,
        out_type=[jax.ShapeDtypeStruct((_B, _S), jnp.float32)] * 3,
        mesh=mesh,
        compiler_params=cparams,
        scratch_types=[
            pltpu.VMEM((_N // 4,), jnp.float32),
            pltpu.VMEM((_N // 4,), jnp.float32),
            pltpu.VMEM((_N // 4,), jnp.float32),
            pltpu.VMEM((_N // 4,), jnp.float32),
            pltpu.VMEM((_S,), jnp.float32),
            pltpu.VMEM((_S,), jnp.float32),
            pltpu.VMEM((_S,), jnp.float32),
            pltpu.VMEM((_L,), jnp.float32),
            pltpu.VMEM((4, _L), jnp.float32),
            pltpu.VMEM_SHARED((2, _L, _L), jnp.float32),
        ],
    )
    ball = pl.kernel(
        _ball_body,
        out_type=jax.ShapeDtypeStruct((_M, _CW), jnp.float32),
        mesh=mesh,
        compiler_params=cparams,
        scratch_types=[
            pltpu.VMEM((_N,), jnp.float32),
            pltpu.VMEM((_N,), jnp.float32),
            pltpu.VMEM((_N,), jnp.float32),
            pltpu.VMEM((_S // 4,), jnp.float32),
            pltpu.VMEM((_S // 4,), jnp.float32),
            pltpu.VMEM((_S // 4,), jnp.float32),
            pltpu.VMEM((_S // 4,), jnp.int32),
            pltpu.VMEM((_S // 4 * _K,), jnp.int32),
            pltpu.VMEM((128, _CW), jnp.float32),
            pltpu.SemaphoreType.DMA,
        ],
    )
    return fps, ball


def _moments_body(x_ref, g_ref, s_ref):
    i = pl.program_id(0)

    @pl.when(i == 0)
    def _():
        g_ref[...] = jnp.zeros_like(g_ref)
        s_ref[...] = jnp.zeros_like(s_ref)

    x = x_ref[...]
    g_ref[...] += lax.dot_general(x, x, (((0,), (0,)), ((), ())),
                                  preferred_element_type=jnp.float32)
    s_ref[...] += jnp.sum(x, axis=0, keepdims=True)


def _moments(x):
    m, c = x.shape
    return pl.pallas_call(
        _moments_body,
        grid=(m // _TM,),
        in_specs=[pl.BlockSpec((_TM, c), lambda i: (i, 0))],
        out_specs=[pl.BlockSpec((c, c), lambda i: (0, 0)),
                   pl.BlockSpec((1, c), lambda i: (0, 0))],
        out_shape=[jax.ShapeDtypeStruct((c, c), jnp.float32),
                   jax.ShapeDtypeStruct((1, c), jnp.float32)],
    )(x)


def _layer_body(x_ref, w_ref, b_ref, y_ref, g_ref, s_ref):
    i = pl.program_id(0)

    @pl.when(i == 0)
    def _():
        g_ref[...] = jnp.zeros_like(g_ref)
        s_ref[...] = jnp.zeros_like(s_ref)

    y = lax.dot_general(x_ref[...], w_ref[...], (((1,), (0,)), ((), ())),
                        preferred_element_type=jnp.float32)
    y = jnp.maximum(y + b_ref[...], 0.0)
    y_ref[...] = y
    g_ref[...] += lax.dot_general(y, y, (((0,), (0,)), ((), ())),
                                  preferred_element_type=jnp.float32)
    s_ref[...] += jnp.sum(y, axis=0, keepdims=True)


def _layer(x, w, b):
    m, c = x.shape
    o = w.shape[1]
    return pl.pallas_call(
        _layer_body,
        grid=(m // _TM,),
        in_specs=[pl.BlockSpec((_TM, c), lambda i: (i, 0)),
                  pl.BlockSpec((c, o), lambda i: (0, 0)),
                  pl.BlockSpec((1, o), lambda i: (0, 0))],
        out_specs=[pl.BlockSpec((_TM, o), lambda i: (i, 0)),
                   pl.BlockSpec((o, o), lambda i: (0, 0)),
                   pl.BlockSpec((1, o), lambda i: (0, 0))],
        out_shape=[jax.ShapeDtypeStruct((m, o), jnp.float32),
                   jax.ShapeDtypeStruct((o, o), jnp.float32),
                   jax.ShapeDtypeStruct((1, o), jnp.float32)],
    )(x, w, b.reshape(1, o))


def _final_body(x_ref, w_ref, b_ref, o_ref):
    y = lax.dot_general(x_ref[...], w_ref[...], (((1,), (0,)), ((), ())),
                        preferred_element_type=jnp.float32)
    y = jnp.maximum(y + b_ref[...], 0.0)
    parts = [jnp.max(y[j * _K:(j + 1) * _K, :], axis=0, keepdims=True)
             for j in range(_TM // _K)]
    o_ref[...] = jnp.concatenate(parts, axis=0)


def _final(x, w, b):
    m, c = x.shape
    o = w.shape[1]
    return pl.pallas_call(
        _final_body,
        grid=(m // _TM,),
        in_specs=[pl.BlockSpec((_TM, c), lambda i: (i, 0)),
                  pl.BlockSpec((c, o), lambda i: (0, 0)),
                  pl.BlockSpec((1, o), lambda i: (0, 0))],
        out_specs=pl.BlockSpec((_TM // _K, o), lambda i: (i, 0)),
        out_shape=jax.ShapeDtypeStruct((m // _K, o), jnp.float32),
    )(x, w, b.reshape(1, o))


def _fold(G, S, W, b, g, be):
    mu = S[0] / _M
    muW = mu @ W
    mean_y = muW + b
    T = G @ W
    ey2 = jnp.sum(W * T, axis=0) / _M + 2.0 * b * muW + b * b
    var = ey2 - mean_y * mean_y
    scale = g / jnp.sqrt(var + 1e-5)
    return W * scale[None, :], (b - mean_y) * scale + be


def kernel(xyz, points, W0, b0, g0, be0, W1, b1, g1, be1, W2, b2, g2, be2):
    xb = xyz[:, 0, :]
    yb = xyz[:, 1, :]
    zb = xyz[:, 2, :]
    fps_call, ball_call = _sc_kernels()
    cx, cy, cz = fps_call(xb, yb, zb)
    new_xyz = jnp.stack([cx, cy, cz], axis=1)

    R = jnp.concatenate(
        [points.transpose(0, 2, 1), xyz.transpose(0, 2, 1),
         jnp.zeros((_B, _N, _CW - 67), jnp.float32)], axis=-1,
    ).reshape(_B * _N, _CW)
    x0 = ball_call(xb, yb, zb, cx, cy, cz, R)

    W0e = jnp.zeros((_CW, W0.shape[0]), jnp.float32)
    W0e = W0e.at[0:64, :].set(W0[:, 3:67].T)
    W0e = W0e.at[64:67, :].set(W0[:, 0:3].T)

    G0, S0 = _moments(x0)
    W0f, b0f = _fold(G0, S0, W0e, b0, g0, be0)
    x1, G1, S1 = _layer(x0, W0f, b0f)
    W1f, b1f = _fold(G1, S1, W1.T, b1, g1, be1)
    x2, G2, S2 = _layer(x1, W1f, b1f)
    W2f, b2f = _fold(G2, S2, W2.T, b2, g2, be2)
    feats = _final(x2, W2f, b2f)
    new_features = feats.reshape(_B, _S, W2.shape[0]).transpose(0, 2, 1)
    return new_xyz, new_features


# speedup vs baseline: 3.7262x; 1.4312x over previous
import functools

import jax
import jax.numpy as jnp
from jax import lax
from jax.experimental import pallas as pl
from jax.experimental.pallas import tpu as pltpu
from jax.experimental.pallas import tpu_sc as plsc

_B = 8
_N = 4096
_S = 1024
_K = 32
_R2 = 0.1 * 0.1
_CW = 80
_M = _B * _S * _K
_TM = 2048
_L = 16

def _fps_body(x_hbm, y_hbm, z_hbm, ocx, ocy, ocz,
              xv, yv, zv, dv, cxv, cyv, czv, pubv, exch, shr):
    c = lax.axis_index("c")
    s = lax.axis_index("s")
    b = c * 4 + s // 4
    q = s % 4
    nq = _N // 4
    qoff = q * nq
    sg = (s // 4) * 4
    lanes = lax.iota(jnp.int32, _L)
    lane0 = lanes == 0
    rowm = lanes & 3
    pltpu.sync_copy(x_hbm.at[b, pl.ds(qoff, nq)], xv)
    pltpu.sync_copy(y_hbm.at[b, pl.ds(qoff, nq)], yv)
    pltpu.sync_copy(z_hbm.at[b, pl.ds(qoff, nq)], zv)
    big = jnp.full((_L,), 1e10, jnp.float32)

    def initc(i, carry):
        dv[pl.ds(i * _L, _L)] = big
        return carry

    lax.fori_loop(0, nq // _L, initc, 0)

    def exchange(par, val, idxf, xc, yc, zc):
        p = jnp.broadcast_to(val, (_L,)).astype(jnp.float32)
        p = jnp.where(lanes == 1, idxf, p)
        p = jnp.where(lanes == 2, xc, p)
        p = jnp.where(lanes == 3, yc, p)
        p = jnp.where(lanes == 4, zc, p)
        pubv[...] = p
        pltpu.sync_copy(pubv, shr.at[par, s])
        plsc.subcore_barrier()
        pltpu.sync_copy(shr.at[par, pl.ds(sg, 4)], exch)
        vals = plsc.load_gather(exch, [rowm, jnp.full((_L,), 0, jnp.int32)])
        m = jnp.max(vals)
        idp = plsc.load_gather(exch, [rowm, jnp.full((_L,), 1, jnp.int32)])
        cand = jnp.where(vals == m, idp, jnp.float32(1e30))
        bidf = jnp.min(cand)
        wm = jnp.logical_and(vals == m, idp == bidf)
        xs = plsc.load_gather(exch, [rowm, jnp.full((_L,), 2, jnp.int32)])
        ys = plsc.load_gather(exch, [rowm, jnp.full((_L,), 3, jnp.int32)])
        zs = plsc.load_gather(exch, [rowm, jnp.full((_L,), 4, jnp.int32)])
        neg = jnp.float32(-1e30)
        nx = jnp.max(jnp.where(wm, xs, neg))
        ny = jnp.max(jnp.where(wm, ys, neg))
        nz = jnp.max(jnp.where(wm, zs, neg))
        return nx, ny, nz

    v0 = jnp.where(q == 0, jnp.float32(1.0), jnp.float32(0.0))
    x00 = xv[pl.ds(0, _L)][0]
    y00 = yv[pl.ds(0, _L)][0]
    z00 = zv[pl.ds(0, _L)][0]
    cx0, cy0, cz0 = exchange(1, v0, jnp.float32(0.0), x00, y00, z00)

    def step(i, carry):
        cx, cy, cz = carry

        @pl.when(q == 0)
        def _():
            ivec = jnp.full((_L,), i, jnp.int32)
            plsc.store_scatter(cxv, [ivec], jnp.full((_L,), cx, jnp.float32),
                               mask=lane0)
            plsc.store_scatter(cyv, [ivec], jnp.full((_L,), cy, jnp.float32),
                               mask=lane0)
            plsc.store_scatter(czv, [ivec], jnp.full((_L,), cz, jnp.float32),
                               mask=lane0)

        def chunk(cc, car):
            mx, bi = car
            for k in range(4):
                off = cc * (4 * _L) + k * _L
                sl = pl.ds(off, _L)
                dx = xv[sl] - cx
                dy = yv[sl] - cy
                dz = zv[sl] - cz
                d = dx * dx + dy * dy
                d = d + dz * dz
                dn = jnp.minimum(dv[sl], d)
                dv[sl] = dn
                gt = dn > mx
                mx = jnp.where(gt, dn, mx)
                bi = jnp.where(gt, off + lanes, bi)
            return mx, bi

        mx, bi = lax.fori_loop(
            0, nq // (4 * _L), chunk,
            (jnp.full((_L,), -1.0, jnp.float32), jnp.zeros((_L,), jnp.int32)))
        m = jnp.max(mx)
        cand = jnp.where(mx == m, bi, nq)
        bl = jnp.min(cand)
        cvx = plsc.load_gather(xv, [jnp.full((_L,), bl, jnp.int32)])
        cvy = plsc.load_gather(yv, [jnp.full((_L,), bl, jnp.int32)])
        cvz = plsc.load_gather(zv, [jnp.full((_L,), bl, jnp.int32)])
        bgf = (qoff + bl).astype(jnp.float32)
        return exchange(i & 1, m, bgf, cvx, cvy, cvz)

    lax.fori_loop(0, _S, step, (cx0, cy0, cz0))

    @pl.when(q == 0)
    def _():
        pltpu.sync_copy(cxv, ocx.at[b])
        pltpu.sync_copy(cyv, ocy.at[b])
        pltpu.sync_copy(czv, ocz.at[b])


def _ball_body(x_hbm, y_hbm, z_hbm, cx_hbm, cy_hbm, cz_hbm, r_hbm, x0_hbm,
               xv, yv, zv, ccx, ccy, ccz, cntb, idxb, rows, sem):
    w = lax.axis_index("s") * 2 + lax.axis_index("c")
    b = w // 4
    q = w % 4
    sc = _S // 4
    pltpu.sync_copy(x_hbm.at[b], xv)
    pltpu.sync_copy(y_hbm.at[b], yv)
    pltpu.sync_copy(z_hbm.at[b], zv)
    pltpu.sync_copy(cx_hbm.at[b, pl.ds(q * sc, sc)], ccx)
    pltpu.sync_copy(cy_hbm.at[b, pl.ds(q * sc, sc)], ccy)
    pltpu.sync_copy(cz_hbm.at[b, pl.ds(q * sc, sc)], ccz)
    lanes = lax.iota(jnp.int32, _L)
    gb = b * _N
    gbv = jnp.full((_L,), gb, jnp.int32)

    def group(g, carry):
        slc = pl.ds(g * _L, _L)
        cxg = ccx[slc]
        cyg = ccy[slc]
        czg = ccz[slc]
        slotbase = (g * _L + lanes) * _K
        plsc.store_scatter(idxb, [slotbase], gbv)

        def pchunk(c, cnt):
            sl = pl.ds(c * _L, _L)
            xs = xv[sl]
            ys = yv[sl]
            zs = zv[sl]
            for k in range(_L):
                dx = cxg - xs[k]
                dy = cyg - ys[k]
                dz = czg - zs[k]
                d = dx * dx + dy * dy
                d = d + dz * dz
                msk = d < _R2
                okm = jnp.logical_and(msk, cnt < _K)
                gidx = jnp.full((_L,), gb + c * _L + k, jnp.int32)
                plsc.store_scatter(idxb, [slotbase + cnt], gidx, mask=okm)
                cnt = cnt + msk.astype(jnp.int32)
            return cnt

        cnt = lax.fori_loop(0, _N // _L, pchunk, jnp.zeros((_L,), jnp.int32))
        cntb[slc] = jnp.minimum(cnt, _K)
        return carry

    lax.fori_loop(0, sc // _L, group, 0)

    def block(blk, carry):
        for t in range(4):
            cl = blk * 4 + t
            clv = jnp.full((_L,), cl, jnp.int32)
            cntv = plsc.load_gather(cntb, [clv])
            firstv = plsc.load_gather(idxb, [clv * _K])
            for j in range(_K // _L):
                sl = pl.ds(cl * _K + j * _L, _L)
                keep = (j * _L + lanes) < cntv
                idxb[sl] = jnp.where(keep, idxb[sl], firstv)
        pltpu.async_copy(r_hbm.at[idxb.at[pl.ds(blk * 128, 128)]],
                         rows, sem).wait()
        for t in range(4):
            cl = blk * 4 + t
            clv = jnp.full((_L,), cl, jnp.int32)
            ctx = plsc.load_gather(ccx, [clv])
            cty = plsc.load_gather(ccy, [clv])
            ctz = plsc.load_gather(ccz, [clv])
            for coord, ctv in ((64, ctx), (65, cty), (66, ctz)):
                colv = jnp.full((_L,), coord, jnp.int32)
                for j in range(_K // _L):
                    ridx = t * _K + j * _L + lanes
                    vals = plsc.load_gather(rows, [ridx, colv])
                    plsc.store_scatter(rows, [ridx, colv], vals - ctv)
        gc = (b * _S + q * sc + blk * 4) * _K
        pltpu.sync_copy(rows, x0_hbm.at[pl.ds(gc, 128)])
        return carry

    lax.fori_loop(0, sc // 4, block, 0)


@functools.lru_cache(maxsize=None)
def _sc_kernels():
    mesh = plsc.VectorSubcoreMesh(core_axis_name="c", subcore_axis_name="s",
                                  num_cores=2, num_subcores=16)
    cparams = pltpu.CompilerParams(needs_layout_passes=False,
                                   use_tc_tiling_on_sc=False)
    fps = pl.kernel(
        _fps_body,
        out_type=[jax.ShapeDtypeStruct((_B, _S), jnp.float32)] * 3,
        mesh=mesh,
        compiler_params=cparams,
        scratch_types=[
            pltpu.VMEM((_N // 4,), jnp.float32),
            pltpu.VMEM((_N // 4,), jnp.float32),
            pltpu.VMEM((_N // 4,), jnp.float32),
            pltpu.VMEM((_N // 4,), jnp.float32),
            pltpu.VMEM((_S,), jnp.float32),
            pltpu.VMEM((_S,), jnp.float32),
            pltpu.VMEM((_S,), jnp.float32),
            pltpu.VMEM((_L,), jnp.float32),
            pltpu.VMEM((4, _L), jnp.float32),
            pltpu.VMEM_SHARED((2, _L, _L), jnp.float32),
        ],
    )
    ball = pl.kernel(
        _ball_body,
        out_type=jax.ShapeDtypeStruct((_M, _CW), jnp.float32),
        mesh=mesh,
        compiler_params=cparams,
        scratch_types=[
            pltpu.VMEM((_N,), jnp.float32),
            pltpu.VMEM((_N,), jnp.float32),
            pltpu.VMEM((_N,), jnp.float32),
            pltpu.VMEM((_S // 4,), jnp.float32),
            pltpu.VMEM((_S // 4,), jnp.float32),
            pltpu.VMEM((_S // 4,), jnp.float32),
            pltpu.VMEM((_S // 4,), jnp.int32),
            pltpu.VMEM((_S // 4 * _K,), jnp.int32),
            pltpu.VMEM((128, _CW), jnp.float32),
            pltpu.SemaphoreType.DMA,
        ],
    )
    return fps, ball


def _moments_body(x_ref, g_ref, s_ref):
    i = pl.program_id(0)

    @pl.when(i == 0)
    def _():
        g_ref[...] = jnp.zeros_like(g_ref)
        s_ref[...] = jnp.zeros_like(s_ref)

    x = x_ref[...]
    g_ref[...] += lax.dot_general(x, x, (((0,), (0,)), ((), ())),
                                  preferred_element_type=jnp.float32)
    s_ref[...] += jnp.sum(x, axis=0, keepdims=True)


def _moments(x):
    m, c = x.shape
    return pl.pallas_call(
        _moments_body,
        grid=(m // _TM,),
        in_specs=[pl.BlockSpec((_TM, c), lambda i: (i, 0))],
        out_specs=[pl.BlockSpec((c, c), lambda i: (0, 0)),
                   pl.BlockSpec((1, c), lambda i: (0, 0))],
        out_shape=[jax.ShapeDtypeStruct((c, c), jnp.float32),
                   jax.ShapeDtypeStruct((1, c), jnp.float32)],
    )(x)


def _layer_body(x_ref, w_ref, b_ref, y_ref, g_ref, s_ref):
    i = pl.program_id(0)

    @pl.when(i == 0)
    def _():
        g_ref[...] = jnp.zeros_like(g_ref)
        s_ref[...] = jnp.zeros_like(s_ref)

    y = lax.dot_general(x_ref[...], w_ref[...], (((1,), (0,)), ((), ())),
                        preferred_element_type=jnp.float32)
    y = jnp.maximum(y + b_ref[...], 0.0)
    y_ref[...] = y
    g_ref[...] += lax.dot_general(y, y, (((0,), (0,)), ((), ())),
                                  preferred_element_type=jnp.float32)
    s_ref[...] += jnp.sum(y, axis=0, keepdims=True)


def _layer(x, w, b):
    m, c = x.shape
    o = w.shape[1]
    return pl.pallas_call(
        _layer_body,
        grid=(m // _TM,),
        in_specs=[pl.BlockSpec((_TM, c), lambda i: (i, 0)),
                  pl.BlockSpec((c, o), lambda i: (0, 0)),
                  pl.BlockSpec((1, o), lambda i: (0, 0))],
        out_specs=[pl.BlockSpec((_TM, o), lambda i: (i, 0)),
                   pl.BlockSpec((o, o), lambda i: (0, 0)),
                   pl.BlockSpec((1, o), lambda i: (0, 0))],
        out_shape=[jax.ShapeDtypeStruct((m, o), jnp.float32),
                   jax.ShapeDtypeStruct((o, o), jnp.float32),
                   jax.ShapeDtypeStruct((1, o), jnp.float32)],
    )(x, w, b.reshape(1, o))


def _final_body(x_ref, w_ref, b_ref, o_ref):
    y = lax.dot_general(x_ref[...], w_ref[...], (((1,), (0,)), ((), ())),
                        preferred_element_type=jnp.float32)
    y = jnp.maximum(y + b_ref[...], 0.0)
    parts = [jnp.max(y[j * _K:(j + 1) * _K, :], axis=0, keepdims=True)
             for j in range(_TM // _K)]
    o_ref[...] = jnp.concatenate(parts, axis=0)


def _final(x, w, b):
    m, c = x.shape
    o = w.shape[1]
    return pl.pallas_call(
        _final_body,
        grid=(m // _TM,),
        in_specs=[pl.BlockSpec((_TM, c), lambda i: (i, 0)),
                  pl.BlockSpec((c, o), lambda i: (0, 0)),
                  pl.BlockSpec((1, o), lambda i: (0, 0))],
        out_specs=pl.BlockSpec((_TM // _K, o), lambda i: (i, 0)),
        out_shape=jax.ShapeDtypeStruct((m // _K, o), jnp.float32),
    )(x, w, b.reshape(1, o))


def _fold(G, S, W, b, g, be):
    mu = S[0] / _M
    muW = mu @ W
    mean_y = muW + b
    T = G @ W
    ey2 = jnp.sum(W * T, axis=0) / _M + 2.0 * b * muW + b * b
    var = ey2 - mean_y * mean_y
    scale = g / jnp.sqrt(var + 1e-5)
    return W * scale[None, :], (b - mean_y) * scale + be


def kernel(xyz, points, W0, b0, g0, be0, W1, b1, g1, be1, W2, b2, g2, be2):
    xb = xyz[:, 0, :]
    yb = xyz[:, 1, :]
    zb = xyz[:, 2, :]
    fps_call, ball_call = _sc_kernels()
    cx, cy, cz = fps_call(xb, yb, zb)
    new_xyz = jnp.stack([cx, cy, cz], axis=1)

    R = jnp.concatenate(
        [points.transpose(0, 2, 1), xyz.transpose(0, 2, 1),
         jnp.zeros((_B, _N, _CW - 67), jnp.float32)], axis=-1,
    ).reshape(_B * _N, _CW)
    x0 = ball_call(xb, yb, zb, cx, cy, cz, R)

    W0e = jnp.zeros((_CW, W0.shape[0]), jnp.float32)
    W0e = W0e.at[0:64, :].set(W0[:, 3:67].T)
    W0e = W0e.at[64:67, :].set(W0[:, 0:3].T)

    G0, S0 = _moments(x0)
    W0f, b0f = _fold(G0, S0, W0e, b0, g0, be0)
    x1, G1, S1 = _layer(x0, W0f, b0f)
    W1f, b1f = _fold(G1, S1, W1.T, b1, g1, be1)
    x2, G2, S2 = _layer(x1, W1f, b1f)
    W2f, b2f = _fold(G2, S2, W2.T, b2, g2, be2)
    feats = _final(x2, W2f, b2f)
    new_features = feats.reshape(_B, _S, W2.shape[0]).transpose(0, 2, 1)
    return new_xyz, new_features


# speedup vs baseline: 4.1989x; 1.1269x over previous
import functools

import jax
import jax.numpy as jnp
from jax import lax
from jax.experimental import pallas as pl
from jax.experimental.pallas import tpu as pltpu
from jax.experimental.pallas import tpu_sc as plsc

_B = 8
_N = 4096
_S = 1024
_K = 32
_R2 = 0.1 * 0.1
_CW = 80
_M = _B * _S * _K
_TM = 8192
_L = 16

def _fps_body(x_hbm, y_hbm, z_hbm, ocx, ocy, ocz,
              xv, yv, zv, dv, cxv, cyv, czv, pubv, exch, shr):
    c = lax.axis_index("c")
    s = lax.axis_index("s")
    b = c * 4 + s // 4
    q = s % 4
    nq = _N // 4
    qoff = q * nq
    sg = (s // 4) * 4
    lanes = lax.iota(jnp.int32, _L)
    lane0 = lanes == 0
    rowm = lanes & 3
    pltpu.sync_copy(x_hbm.at[b, pl.ds(qoff, nq)], xv)
    pltpu.sync_copy(y_hbm.at[b, pl.ds(qoff, nq)], yv)
    pltpu.sync_copy(z_hbm.at[b, pl.ds(qoff, nq)], zv)
    big = jnp.full((_L,), 1e10, jnp.float32)

    def initc(i, carry):
        dv[pl.ds(i * _L, _L)] = big
        return carry

    lax.fori_loop(0, nq // _L, initc, 0)

    def exchange(par, val, idxf, xc, yc, zc):
        p = jnp.broadcast_to(val, (_L,)).astype(jnp.float32)
        p = jnp.where(lanes == 1, idxf, p)
        p = jnp.where(lanes == 2, xc, p)
        p = jnp.where(lanes == 3, yc, p)
        p = jnp.where(lanes == 4, zc, p)
        pubv[...] = p
        pltpu.sync_copy(pubv, shr.at[par, s])
        plsc.subcore_barrier()
        pltpu.sync_copy(shr.at[par, pl.ds(sg, 4)], exch)
        vals = plsc.load_gather(exch, [rowm, jnp.full((_L,), 0, jnp.int32)])
        m = jnp.max(vals)
        idp = plsc.load_gather(exch, [rowm, jnp.full((_L,), 1, jnp.int32)])
        cand = jnp.where(vals == m, idp, jnp.float32(1e30))
        bidf = jnp.min(cand)
        wm = jnp.logical_and(vals == m, idp == bidf)
        xs = plsc.load_gather(exch, [rowm, jnp.full((_L,), 2, jnp.int32)])
        ys = plsc.load_gather(exch, [rowm, jnp.full((_L,), 3, jnp.int32)])
        zs = plsc.load_gather(exch, [rowm, jnp.full((_L,), 4, jnp.int32)])
        neg = jnp.float32(-1e30)
        nx = jnp.max(jnp.where(wm, xs, neg))
        ny = jnp.max(jnp.where(wm, ys, neg))
        nz = jnp.max(jnp.where(wm, zs, neg))
        return nx, ny, nz

    v0 = jnp.where(q == 0, jnp.float32(1.0), jnp.float32(0.0))
    x00 = xv[pl.ds(0, _L)][0]
    y00 = yv[pl.ds(0, _L)][0]
    z00 = zv[pl.ds(0, _L)][0]
    cx0, cy0, cz0 = exchange(1, v0, jnp.float32(0.0), x00, y00, z00)

    def step(i, carry):
        cx, cy, cz = carry

        @pl.when(q == 0)
        def _():
            ivec = jnp.full((_L,), i, jnp.int32)
            plsc.store_scatter(cxv, [ivec], jnp.full((_L,), cx, jnp.float32),
                               mask=lane0)
            plsc.store_scatter(cyv, [ivec], jnp.full((_L,), cy, jnp.float32),
                               mask=lane0)
            plsc.store_scatter(czv, [ivec], jnp.full((_L,), cz, jnp.float32),
                               mask=lane0)

        def chunk(cc, car):
            mx, bi = car
            for k in range(4):
                off = cc * (4 * _L) + k * _L
                sl = pl.ds(off, _L)
                dx = xv[sl] - cx
                dy = yv[sl] - cy
                dz = zv[sl] - cz
                d = dx * dx + dy * dy
                d = d + dz * dz
                dn = jnp.minimum(dv[sl], d)
                dv[sl] = dn
                gt = dn > mx
                mx = jnp.where(gt, dn, mx)
                bi = jnp.where(gt, off + lanes, bi)
            return mx, bi

        mx, bi = lax.fori_loop(
            0, nq // (4 * _L), chunk,
            (jnp.full((_L,), -1.0, jnp.float32), jnp.zeros((_L,), jnp.int32)))
        m = jnp.max(mx)
        cand = jnp.where(mx == m, bi, nq)
        bl = jnp.min(cand)
        cvx = plsc.load_gather(xv, [jnp.full((_L,), bl, jnp.int32)])
        cvy = plsc.load_gather(yv, [jnp.full((_L,), bl, jnp.int32)])
        cvz = plsc.load_gather(zv, [jnp.full((_L,), bl, jnp.int32)])
        bgf = (qoff + bl).astype(jnp.float32)
        return exchange(i & 1, m, bgf, cvx, cvy, cvz)

    lax.fori_loop(0, _S, step, (cx0, cy0, cz0))

    @pl.when(q == 0)
    def _():
        pltpu.sync_copy(cxv, ocx.at[b])
        pltpu.sync_copy(cyv, ocy.at[b])
        pltpu.sync_copy(czv, ocz.at[b])


def _ball_body(x_hbm, y_hbm, z_hbm, cx_hbm, cy_hbm, cz_hbm, r_hbm, x0_hbm,
               xv, yv, zv, ccx, ccy, ccz, cntb, idxb, rows, sem):
    w = lax.axis_index("s") * 2 + lax.axis_index("c")
    b = w // 4
    q = w % 4
    sc = _S // 4
    pltpu.sync_copy(x_hbm.at[b], xv)
    pltpu.sync_copy(y_hbm.at[b], yv)
    pltpu.sync_copy(z_hbm.at[b], zv)
    pltpu.sync_copy(cx_hbm.at[b, pl.ds(q * sc, sc)], ccx)
    pltpu.sync_copy(cy_hbm.at[b, pl.ds(q * sc, sc)], ccy)
    pltpu.sync_copy(cz_hbm.at[b, pl.ds(q * sc, sc)], ccz)
    lanes = lax.iota(jnp.int32, _L)
    gb = b * _N
    gbv = jnp.full((_L,), gb, jnp.int32)

    def group(g, carry):
        slc = pl.ds(g * _L, _L)
        cxg = ccx[slc]
        cyg = ccy[slc]
        czg = ccz[slc]
        slotbase = (g * _L + lanes) * _K
        plsc.store_scatter(idxb, [slotbase], gbv)

        def pchunk(c, cnt):
            sl = pl.ds(c * _L, _L)
            xs = xv[sl]
            ys = yv[sl]
            zs = zv[sl]
            for k in range(_L):
                dx = cxg - xs[k]
                dy = cyg - ys[k]
                dz = czg - zs[k]
                d = dx * dx + dy * dy
                d = d + dz * dz
                msk = d < _R2
                okm = jnp.logical_and(msk, cnt < _K)
                gidx = jnp.full((_L,), gb + c * _L + k, jnp.int32)
                plsc.store_scatter(idxb, [slotbase + cnt], gidx, mask=okm)
                cnt = cnt + msk.astype(jnp.int32)
            return cnt

        cnt = lax.fori_loop(0, _N // _L, pchunk, jnp.zeros((_L,), jnp.int32))
        cntb[slc] = jnp.minimum(cnt, _K)
        return carry

    lax.fori_loop(0, sc // _L, group, 0)

    def block(blk, carry):
        for t in range(4):
            cl = blk * 4 + t
            clv = jnp.full((_L,), cl, jnp.int32)
            cntv = plsc.load_gather(cntb, [clv])
            firstv = plsc.load_gather(idxb, [clv * _K])
            for j in range(_K // _L):
                sl = pl.ds(cl * _K + j * _L, _L)
                keep = (j * _L + lanes) < cntv
                idxb[sl] = jnp.where(keep, idxb[sl], firstv)
        pltpu.async_copy(r_hbm.at[idxb.at[pl.ds(blk * 128, 128)]],
                         rows, sem).wait()
        for t in range(4):
            cl = blk * 4 + t
            clv = jnp.full((_L,), cl, jnp.int32)
            ctx = plsc.load_gather(ccx, [clv])
            cty = plsc.load_gather(ccy, [clv])
            ctz = plsc.load_gather(ccz, [clv])
            for coord, ctv in ((64, ctx), (65, cty), (66, ctz)):
                colv = jnp.full((_L,), coord, jnp.int32)
                for j in range(_K // _L):
                    ridx = t * _K + j * _L + lanes
                    vals = plsc.load_gather(rows, [ridx, colv])
                    plsc.store_scatter(rows, [ridx, colv], vals - ctv)
        gc = (b * _S + q * sc + blk * 4) * _K
        pltpu.sync_copy(rows, x0_hbm.at[pl.ds(gc, 128)])
        return carry

    lax.fori_loop(0, sc // 4, block, 0)


@functools.lru_cache(maxsize=None)
def _sc_kernels():
    mesh = plsc.VectorSubcoreMesh(core_axis_name="c", subcore_axis_name="s",
                                  num_cores=2, num_subcores=16)
    cparams = pltpu.CompilerParams(needs_layout_passes=False,
                                   use_tc_tiling_on_sc=False)
    fps = pl.kernel(
        _fps_body,
        out_type=[jax.ShapeDtypeStruct((_B, _S), jnp.float32)] * 3,
        mesh=mesh,
        compiler_params=cparams,
        scratch_types=[
            pltpu.VMEM((_N // 4,), jnp.float32),
            pltpu.VMEM((_N // 4,), jnp.float32),
            pltpu.VMEM((_N // 4,), jnp.float32),
            pltpu.VMEM((_N // 4,), jnp.float32),
            pltpu.VMEM((_S,), jnp.float32),
            pltpu.VMEM((_S,), jnp.float32),
            pltpu.VMEM((_S,), jnp.float32),
            pltpu.VMEM((_L,), jnp.float32),
            pltpu.VMEM((4, _L), jnp.float32),
            pltpu.VMEM_SHARED((2, _L, _L), jnp.float32),
        ],
    )
    ball = pl.kernel(
        _ball_body,
        out_type=jax.ShapeDtypeStruct((_M, _CW), jnp.float32),
        mesh=mesh,
        compiler_params=cparams,
        scratch_types=[
            pltpu.VMEM((_N,), jnp.float32),
            pltpu.VMEM((_N,), jnp.float32),
            pltpu.VMEM((_N,), jnp.float32),
            pltpu.VMEM((_S // 4,), jnp.float32),
            pltpu.VMEM((_S // 4,), jnp.float32),
            pltpu.VMEM((_S // 4,), jnp.float32),
            pltpu.VMEM((_S // 4,), jnp.int32),
            pltpu.VMEM((_S // 4 * _K,), jnp.int32),
            pltpu.VMEM((128, _CW), jnp.float32),
            pltpu.SemaphoreType.DMA,
        ],
    )
    return fps, ball


def _moments_body(x_ref, g_ref, s_ref):
    i = pl.program_id(0)

    @pl.when(i == 0)
    def _():
        g_ref[...] = jnp.zeros_like(g_ref)
        s_ref[...] = jnp.zeros_like(s_ref)

    x = x_ref[...]
    g_ref[...] += lax.dot_general(x, x, (((0,), (0,)), ((), ())),
                                  preferred_element_type=jnp.float32)
    s_ref[...] += jnp.sum(x, axis=0, keepdims=True)


def _moments(x):
    m, c = x.shape
    return pl.pallas_call(
        _moments_body,
        grid=(m // _TM,),
        in_specs=[pl.BlockSpec((_TM, c), lambda i: (i, 0))],
        out_specs=[pl.BlockSpec((c, c), lambda i: (0, 0)),
                   pl.BlockSpec((1, c), lambda i: (0, 0))],
        out_shape=[jax.ShapeDtypeStruct((c, c), jnp.float32),
                   jax.ShapeDtypeStruct((1, c), jnp.float32)],
    )(x)


def _layer_body(x_ref, w_ref, b_ref, y_ref, g_ref, s_ref):
    i = pl.program_id(0)

    @pl.when(i == 0)
    def _():
        g_ref[...] = jnp.zeros_like(g_ref)
        s_ref[...] = jnp.zeros_like(s_ref)

    y = lax.dot_general(x_ref[...], w_ref[...], (((1,), (0,)), ((), ())),
                        preferred_element_type=jnp.float32)
    y = jnp.maximum(y + b_ref[...], 0.0)
    y_ref[...] = y
    g_ref[...] += lax.dot_general(y, y, (((0,), (0,)), ((), ())),
                                  preferred_element_type=jnp.float32)
    s_ref[...] += jnp.sum(y, axis=0, keepdims=True)


def _layer(x, w, b):
    m, c = x.shape
    o = w.shape[1]
    return pl.pallas_call(
        _layer_body,
        grid=(m // _TM,),
        in_specs=[pl.BlockSpec((_TM, c), lambda i: (i, 0)),
                  pl.BlockSpec((c, o), lambda i: (0, 0)),
                  pl.BlockSpec((1, o), lambda i: (0, 0))],
        out_specs=[pl.BlockSpec((_TM, o), lambda i: (i, 0)),
                   pl.BlockSpec((o, o), lambda i: (0, 0)),
                   pl.BlockSpec((1, o), lambda i: (0, 0))],
        out_shape=[jax.ShapeDtypeStruct((m, o), jnp.float32),
                   jax.ShapeDtypeStruct((o, o), jnp.float32),
                   jax.ShapeDtypeStruct((1, o), jnp.float32)],
    )(x, w, b.reshape(1, o))


def _final_body(x_ref, w_ref, b_ref, o_ref):
    y = lax.dot_general(x_ref[...], w_ref[...], (((1,), (0,)), ((), ())),
                        preferred_element_type=jnp.float32)
    y = jnp.maximum(y + b_ref[...], 0.0)
    parts = [jnp.max(y[j * _K:(j + 1) * _K, :], axis=0, keepdims=True)
             for j in range(_TM // _K)]
    o_ref[...] = jnp.concatenate(parts, axis=0)


def _final(x, w, b):
    m, c = x.shape
    o = w.shape[1]
    return pl.pallas_call(
        _final_body,
        grid=(m // _TM,),
        in_specs=[pl.BlockSpec((_TM, c), lambda i: (i, 0)),
                  pl.BlockSpec((c, o), lambda i: (0, 0)),
                  pl.BlockSpec((1, o), lambda i: (0, 0))],
        out_specs=pl.BlockSpec((_TM // _K, o), lambda i: (i, 0)),
        out_shape=jax.ShapeDtypeStruct((m // _K, o), jnp.float32),
    )(x, w, b.reshape(1, o))


def _fold(G, S, W, b, g, be):
    mu = S[0] / _M
    muW = mu @ W
    mean_y = muW + b
    T = G @ W
    ey2 = jnp.sum(W * T, axis=0) / _M + 2.0 * b * muW + b * b
    var = ey2 - mean_y * mean_y
    scale = g / jnp.sqrt(var + 1e-5)
    return W * scale[None, :], (b - mean_y) * scale + be


def kernel(xyz, points, W0, b0, g0, be0, W1, b1, g1, be1, W2, b2, g2, be2):
    xb = xyz[:, 0, :]
    yb = xyz[:, 1, :]
    zb = xyz[:, 2, :]
    fps_call, ball_call = _sc_kernels()
    cx, cy, cz = fps_call(xb, yb, zb)
    new_xyz = jnp.stack([cx, cy, cz], axis=1)

    R = jnp.concatenate(
        [points.transpose(0, 2, 1), xyz.transpose(0, 2, 1),
         jnp.zeros((_B, _N, _CW - 67), jnp.float32)], axis=-1,
    ).reshape(_B * _N, _CW)
    x0 = ball_call(xb, yb, zb, cx, cy, cz, R)

    W0e = jnp.zeros((_CW, W0.shape[0]), jnp.float32)
    W0e = W0e.at[0:64, :].set(W0[:, 3:67].T)
    W0e = W0e.at[64:67, :].set(W0[:, 0:3].T)

    G0, S0 = _moments(x0)
    W0f, b0f = _fold(G0, S0, W0e, b0, g0, be0)
    x1, G1, S1 = _layer(x0, W0f, b0f)
    W1f, b1f = _fold(G1, S1, W1.T, b1, g1, be1)
    x2, G2, S2 = _layer(x1, W1f, b1f)
    W2f, b2f = _fold(G2, S2, W2.T, b2, g2, be2)
    feats = _final(x2, W2f, b2f)
    new_features = feats.reshape(_B, _S, W2.shape[0]).transpose(0, 2, 1)
    return new_xyz, new_features


# speedup vs baseline: 4.2457x; 1.0111x over previous
import functools

import jax
import jax.numpy as jnp
from jax import lax
from jax.experimental import pallas as pl
from jax.experimental.pallas import tpu as pltpu
from jax.experimental.pallas import tpu_sc as plsc

_B = 8
_N = 4096
_S = 1024
_K = 32
_R2 = 0.1 * 0.1
_CW = 80
_M = _B * _S * _K
_TM = 16384
_L = 16

def _fps_body(x_hbm, y_hbm, z_hbm, ocx, ocy, ocz,
              xv, yv, zv, dv, cxv, cyv, czv, pubv, exch, shr):
    c = lax.axis_index("c")
    s = lax.axis_index("s")
    b = c * 4 + s // 4
    q = s % 4
    nq = _N // 4
    qoff = q * nq
    sg = (s // 4) * 4
    lanes = lax.iota(jnp.int32, _L)
    lane0 = lanes == 0
    rowm = lanes & 3
    pltpu.sync_copy(x_hbm.at[b, pl.ds(qoff, nq)], xv)
    pltpu.sync_copy(y_hbm.at[b, pl.ds(qoff, nq)], yv)
    pltpu.sync_copy(z_hbm.at[b, pl.ds(qoff, nq)], zv)
    big = jnp.full((_L,), 1e10, jnp.float32)

    def initc(i, carry):
        dv[pl.ds(i * _L, _L)] = big
        return carry

    lax.fori_loop(0, nq // _L, initc, 0)

    def exchange(par, val, idxf, xc, yc, zc):
        p = jnp.broadcast_to(val, (_L,)).astype(jnp.float32)
        p = jnp.where(lanes == 1, idxf, p)
        p = jnp.where(lanes == 2, xc, p)
        p = jnp.where(lanes == 3, yc, p)
        p = jnp.where(lanes == 4, zc, p)
        pubv[...] = p
        pltpu.sync_copy(pubv, shr.at[par, s])
        plsc.subcore_barrier()
        pltpu.sync_copy(shr.at[par, pl.ds(sg, 4)], exch)
        vals = plsc.load_gather(exch, [rowm, jnp.full((_L,), 0, jnp.int32)])
        m = jnp.max(vals)
        idp = plsc.load_gather(exch, [rowm, jnp.full((_L,), 1, jnp.int32)])
        cand = jnp.where(vals == m, idp, jnp.float32(1e30))
        bidf = jnp.min(cand)
        wm = jnp.logical_and(vals == m, idp == bidf)
        xs = plsc.load_gather(exch, [rowm, jnp.full((_L,), 2, jnp.int32)])
        ys = plsc.load_gather(exch, [rowm, jnp.full((_L,), 3, jnp.int32)])
        zs = plsc.load_gather(exch, [rowm, jnp.full((_L,), 4, jnp.int32)])
        neg = jnp.float32(-1e30)
        nx = jnp.max(jnp.where(wm, xs, neg))
        ny = jnp.max(jnp.where(wm, ys, neg))
        nz = jnp.max(jnp.where(wm, zs, neg))
        return nx, ny, nz

    v0 = jnp.where(q == 0, jnp.float32(1.0), jnp.float32(0.0))
    x00 = xv[pl.ds(0, _L)][0]
    y00 = yv[pl.ds(0, _L)][0]
    z00 = zv[pl.ds(0, _L)][0]
    cx0, cy0, cz0 = exchange(1, v0, jnp.float32(0.0), x00, y00, z00)

    def step(i, carry):
        cx, cy, cz = carry

        @pl.when(q == 0)
        def _():
            ivec = jnp.full((_L,), i, jnp.int32)
            plsc.store_scatter(cxv, [ivec], jnp.full((_L,), cx, jnp.float32),
                               mask=lane0)
            plsc.store_scatter(cyv, [ivec], jnp.full((_L,), cy, jnp.float32),
                               mask=lane0)
            plsc.store_scatter(czv, [ivec], jnp.full((_L,), cz, jnp.float32),
                               mask=lane0)

        def chunk(cc, car):
            mx, bi = car
            for k in range(4):
                off = cc * (4 * _L) + k * _L
                sl = pl.ds(off, _L)
                dx = xv[sl] - cx
                dy = yv[sl] - cy
                dz = zv[sl] - cz
                d = dx * dx + dy * dy
                d = d + dz * dz
                dn = jnp.minimum(dv[sl], d)
                dv[sl] = dn
                gt = dn > mx
                mx = jnp.where(gt, dn, mx)
                bi = jnp.where(gt, off + lanes, bi)
            return mx, bi

        mx, bi = lax.fori_loop(
            0, nq // (4 * _L), chunk,
            (jnp.full((_L,), -1.0, jnp.float32), jnp.zeros((_L,), jnp.int32)))
        m = jnp.max(mx)
        cand = jnp.where(mx == m, bi, nq)
        bl = jnp.min(cand)
        cvx = plsc.load_gather(xv, [jnp.full((_L,), bl, jnp.int32)])
        cvy = plsc.load_gather(yv, [jnp.full((_L,), bl, jnp.int32)])
        cvz = plsc.load_gather(zv, [jnp.full((_L,), bl, jnp.int32)])
        bgf = (qoff + bl).astype(jnp.float32)
        return exchange(i & 1, m, bgf, cvx, cvy, cvz)

    lax.fori_loop(0, _S, step, (cx0, cy0, cz0))

    @pl.when(q == 0)
    def _():
        pltpu.sync_copy(cxv, ocx.at[b])
        pltpu.sync_copy(cyv, ocy.at[b])
        pltpu.sync_copy(czv, ocz.at[b])


def _ball_body(x_hbm, y_hbm, z_hbm, cx_hbm, cy_hbm, cz_hbm, r_hbm, x0_hbm,
               xv, yv, zv, ccx, ccy, ccz, cntb, idxb, rows, sem):
    w = lax.axis_index("s") * 2 + lax.axis_index("c")
    b = w // 4
    q = w % 4
    sc = _S // 4
    pltpu.sync_copy(x_hbm.at[b], xv)
    pltpu.sync_copy(y_hbm.at[b], yv)
    pltpu.sync_copy(z_hbm.at[b], zv)
    pltpu.sync_copy(cx_hbm.at[b, pl.ds(q * sc, sc)], ccx)
    pltpu.sync_copy(cy_hbm.at[b, pl.ds(q * sc, sc)], ccy)
    pltpu.sync_copy(cz_hbm.at[b, pl.ds(q * sc, sc)], ccz)
    lanes = lax.iota(jnp.int32, _L)
    gb = b * _N
    gbv = jnp.full((_L,), gb, jnp.int32)

    def group(g, carry):
        slc = pl.ds(g * _L, _L)
        cxg = ccx[slc]
        cyg = ccy[slc]
        czg = ccz[slc]
        slotbase = (g * _L + lanes) * _K
        plsc.store_scatter(idxb, [slotbase], gbv)

        def pchunk(c, cnt):
            sl = pl.ds(c * _L, _L)
            xs = xv[sl]
            ys = yv[sl]
            zs = zv[sl]
            for k in range(_L):
                dx = cxg - xs[k]
                dy = cyg - ys[k]
                dz = czg - zs[k]
                d = dx * dx + dy * dy
                d = d + dz * dz
                msk = d < _R2
                okm = jnp.logical_and(msk, cnt < _K)
                gidx = jnp.full((_L,), gb + c * _L + k, jnp.int32)
                plsc.store_scatter(idxb, [slotbase + cnt], gidx, mask=okm)
                cnt = cnt + msk.astype(jnp.int32)
            return cnt

        cnt = lax.fori_loop(0, _N // _L, pchunk, jnp.zeros((_L,), jnp.int32))
        cntb[slc] = jnp.minimum(cnt, _K)
        return carry

    lax.fori_loop(0, sc // _L, group, 0)

    def block(blk, carry):
        for t in range(4):
            cl = blk * 4 + t
            clv = jnp.full((_L,), cl, jnp.int32)
            cntv = plsc.load_gather(cntb, [clv])
            firstv = plsc.load_gather(idxb, [clv * _K])
            for j in range(_K // _L):
                sl = pl.ds(cl * _K + j * _L, _L)
                keep = (j * _L + lanes) < cntv
                idxb[sl] = jnp.where(keep, idxb[sl], firstv)
        pltpu.async_copy(r_hbm.at[idxb.at[pl.ds(blk * 128, 128)]],
                         rows, sem).wait()
        for t in range(4):
            cl = blk * 4 + t
            clv = jnp.full((_L,), cl, jnp.int32)
            ctx = plsc.load_gather(ccx, [clv])
            cty = plsc.load_gather(ccy, [clv])
            ctz = plsc.load_gather(ccz, [clv])
            for coord, ctv in ((64, ctx), (65, cty), (66, ctz)):
                colv = jnp.full((_L,), coord, jnp.int32)
                for j in range(_K // _L):
                    ridx = t * _K + j * _L + lanes
                    vals = plsc.load_gather(rows, [ridx, colv])
                    plsc.store_scatter(rows, [ridx, colv], vals - ctv)
        gc = (b * _S + q * sc + blk * 4) * _K
        pltpu.sync_copy(rows, x0_hbm.at[pl.ds(gc, 128)])
        return carry

    lax.fori_loop(0, sc // 4, block, 0)


@functools.lru_cache(maxsize=None)
def _sc_kernels():
    mesh = plsc.VectorSubcoreMesh(core_axis_name="c", subcore_axis_name="s",
                                  num_cores=2, num_subcores=16)
    cparams = pltpu.CompilerParams(needs_layout_passes=False,
                                   use_tc_tiling_on_sc=False)
    fps = pl.kernel(
        _fps_body,
        out_type=[jax.ShapeDtypeStruct((_B, _S), jnp.float32)] * 3,
        mesh=mesh,
        compiler_params=cparams,
        scratch_types=[
            pltpu.VMEM((_N // 4,), jnp.float32),
            pltpu.VMEM((_N // 4,), jnp.float32),
            pltpu.VMEM((_N // 4,), jnp.float32),
            pltpu.VMEM((_N // 4,), jnp.float32),
            pltpu.VMEM((_S,), jnp.float32),
            pltpu.VMEM((_S,), jnp.float32),
            pltpu.VMEM((_S,), jnp.float32),
            pltpu.VMEM((_L,), jnp.float32),
            pltpu.VMEM((4, _L), jnp.float32),
            pltpu.VMEM_SHARED((2, _L, _L), jnp.float32),
        ],
    )
    ball = pl.kernel(
        _ball_body,
        out_type=jax.ShapeDtypeStruct((_M, _CW), jnp.float32),
        mesh=mesh,
        compiler_params=cparams,
        scratch_types=[
            pltpu.VMEM((_N,), jnp.float32),
            pltpu.VMEM((_N,), jnp.float32),
            pltpu.VMEM((_N,), jnp.float32),
            pltpu.VMEM((_S // 4,), jnp.float32),
            pltpu.VMEM((_S // 4,), jnp.float32),
            pltpu.VMEM((_S // 4,), jnp.float32),
            pltpu.VMEM((_S // 4,), jnp.int32),
            pltpu.VMEM((_S // 4 * _K,), jnp.int32),
            pltpu.VMEM((128, _CW), jnp.float32),
            pltpu.SemaphoreType.DMA,
        ],
    )
    return fps, ball


def _moments_body(x_ref, g_ref, s_ref):
    i = pl.program_id(0)

    @pl.when(i == 0)
    def _():
        g_ref[...] = jnp.zeros_like(g_ref)
        s_ref[...] = jnp.zeros_like(s_ref)

    x = x_ref[...]
    g_ref[...] += lax.dot_general(x, x, (((0,), (0,)), ((), ())),
                                  preferred_element_type=jnp.float32)
    s_ref[...] += jnp.sum(x, axis=0, keepdims=True)


def _moments(x):
    m, c = x.shape
    return pl.pallas_call(
        _moments_body,
        grid=(m // _TM,),
        in_specs=[pl.BlockSpec((_TM, c), lambda i: (i, 0))],
        out_specs=[pl.BlockSpec((c, c), lambda i: (0, 0)),
                   pl.BlockSpec((1, c), lambda i: (0, 0))],
        out_shape=[jax.ShapeDtypeStruct((c, c), jnp.float32),
                   jax.ShapeDtypeStruct((1, c), jnp.float32)],
    )(x)


def _layer_body(x_ref, w_ref, b_ref, y_ref, g_ref, s_ref):
    i = pl.program_id(0)

    @pl.when(i == 0)
    def _():
        g_ref[...] = jnp.zeros_like(g_ref)
        s_ref[...] = jnp.zeros_like(s_ref)

    y = lax.dot_general(x_ref[...], w_ref[...], (((1,), (0,)), ((), ())),
                        preferred_element_type=jnp.float32)
    y = jnp.maximum(y + b_ref[...], 0.0)
    y_ref[...] = y
    g_ref[...] += lax.dot_general(y, y, (((0,), (0,)), ((), ())),
                                  preferred_element_type=jnp.float32)
    s_ref[...] += jnp.sum(y, axis=0, keepdims=True)


def _layer(x, w, b):
    m, c = x.shape
    o = w.shape[1]
    return pl.pallas_call(
        _layer_body,
        grid=(m // _TM,),
        in_specs=[pl.BlockSpec((_TM, c), lambda i: (i, 0)),
                  pl.BlockSpec((c, o), lambda i: (0, 0)),
                  pl.BlockSpec((1, o), lambda i: (0, 0))],
        out_specs=[pl.BlockSpec((_TM, o), lambda i: (i, 0)),
                   pl.BlockSpec((o, o), lambda i: (0, 0)),
                   pl.BlockSpec((1, o), lambda i: (0, 0))],
        out_shape=[jax.ShapeDtypeStruct((m, o), jnp.float32),
                   jax.ShapeDtypeStruct((o, o), jnp.float32),
                   jax.ShapeDtypeStruct((1, o), jnp.float32)],
    )(x, w, b.reshape(1, o))


def _final_body(x_ref, w_ref, b_ref, o_ref):
    y = lax.dot_general(x_ref[...], w_ref[...], (((1,), (0,)), ((), ())),
                        preferred_element_type=jnp.float32)
    y = jnp.maximum(y + b_ref[...], 0.0)
    parts = [jnp.max(y[j * _K:(j + 1) * _K, :], axis=0, keepdims=True)
             for j in range(_TM // _K)]
    o_ref[...] = jnp.concatenate(parts, axis=0)


def _final(x, w, b):
    m, c = x.shape
    o = w.shape[1]
    return pl.pallas_call(
        _final_body,
        grid=(m // _TM,),
        in_specs=[pl.BlockSpec((_TM, c), lambda i: (i, 0)),
                  pl.BlockSpec((c, o), lambda i: (0, 0)),
                  pl.BlockSpec((1, o), lambda i: (0, 0))],
        out_specs=pl.BlockSpec((_TM // _K, o), lambda i: (i, 0)),
        out_shape=jax.ShapeDtypeStruct((m // _K, o), jnp.float32),
    )(x, w, b.reshape(1, o))


def _fold(G, S, W, b, g, be):
    mu = S[0] / _M
    muW = mu @ W
    mean_y = muW + b
    T = G @ W
    ey2 = jnp.sum(W * T, axis=0) / _M + 2.0 * b * muW + b * b
    var = ey2 - mean_y * mean_y
    scale = g / jnp.sqrt(var + 1e-5)
    return W * scale[None, :], (b - mean_y) * scale + be


def kernel(xyz, points, W0, b0, g0, be0, W1, b1, g1, be1, W2, b2, g2, be2):
    xb = xyz[:, 0, :]
    yb = xyz[:, 1, :]
    zb = xyz[:, 2, :]
    fps_call, ball_call = _sc_kernels()
    cx, cy, cz = fps_call(xb, yb, zb)
    new_xyz = jnp.stack([cx, cy, cz], axis=1)

    R = jnp.concatenate(
        [points.transpose(0, 2, 1), xyz.transpose(0, 2, 1),
         jnp.zeros((_B, _N, _CW - 67), jnp.float32)], axis=-1,
    ).reshape(_B * _N, _CW)
    x0 = ball_call(xb, yb, zb, cx, cy, cz, R)

    W0e = jnp.zeros((_CW, W0.shape[0]), jnp.float32)
    W0e = W0e.at[0:64, :].set(W0[:, 3:67].T)
    W0e = W0e.at[64:67, :].set(W0[:, 0:3].T)

    G0, S0 = _moments(x0)
    W0f, b0f = _fold(G0, S0, W0e, b0, g0, be0)
    x1, G1, S1 = _layer(x0, W0f, b0f)
    W1f, b1f = _fold(G1, S1, W1.T, b1, g1, be1)
    x2, G2, S2 = _layer(x1, W1f, b1f)
    W2f, b2f = _fold(G2, S2, W2.T, b2, g2, be2)
    feats = _final(x2, W2f, b2f)
    new_features = feats.reshape(_B, _S, W2.shape[0]).transpose(0, 2, 1)
    return new_xyz, new_features


# speedup vs baseline: 4.2619x; 1.0038x over previous
import functools

import jax
import jax.numpy as jnp
from jax import lax
from jax.experimental import pallas as pl
from jax.experimental.pallas import tpu as pltpu
from jax.experimental.pallas import tpu_sc as plsc

_B = 8
_N = 4096
_S = 1024
_K = 32
_R2 = 0.1 * 0.1
_CW = 80
_M = _B * _S * _K
_TM = 16384
_L = 16

def _fps_body(x_hbm, y_hbm, z_hbm, ocx, ocy, ocz,
              xv, yv, zv, dv, cxv, cyv, czv, pubv, exch, shr):
    c = lax.axis_index("c")
    s = lax.axis_index("s")
    b = c * 4 + s // 4
    q = s % 4
    nq = _N // 4
    qoff = q * nq
    sg = (s // 4) * 4
    lanes = lax.iota(jnp.int32, _L)
    lane0 = lanes == 0
    rowm = lanes & 3
    pltpu.sync_copy(x_hbm.at[b, pl.ds(qoff, nq)], xv)
    pltpu.sync_copy(y_hbm.at[b, pl.ds(qoff, nq)], yv)
    pltpu.sync_copy(z_hbm.at[b, pl.ds(qoff, nq)], zv)
    big = jnp.full((_L,), 1e10, jnp.float32)

    def initc(i, carry):
        dv[pl.ds(i * _L, _L)] = big
        return carry

    lax.fori_loop(0, nq // _L, initc, 0)

    def exchange(par, val, idxf, xc, yc, zc):
        p = jnp.broadcast_to(val, (_L,)).astype(jnp.float32)
        p = jnp.where(lanes == 1, idxf, p)
        p = jnp.where(lanes == 2, xc, p)
        p = jnp.where(lanes == 3, yc, p)
        p = jnp.where(lanes == 4, zc, p)
        pubv[...] = p
        pltpu.sync_copy(pubv, shr.at[par, s])
        plsc.subcore_barrier()
        pltpu.sync_copy(shr.at[par, pl.ds(sg, 4)], exch)
        vals = plsc.load_gather(exch, [rowm, jnp.full((_L,), 0, jnp.int32)])
        m = jnp.max(vals)
        idp = plsc.load_gather(exch, [rowm, jnp.full((_L,), 1, jnp.int32)])
        cand = jnp.where(vals == m, idp, jnp.float32(1e30))
        bidf = jnp.min(cand)
        qw = bidf.astype(jnp.int32) >> 10
        qwv = jnp.full((_L,), qw, jnp.int32)
        nx = plsc.load_gather(exch, [qwv, jnp.full((_L,), 2, jnp.int32)])
        ny = plsc.load_gather(exch, [qwv, jnp.full((_L,), 3, jnp.int32)])
        nz = plsc.load_gather(exch, [qwv, jnp.full((_L,), 4, jnp.int32)])
        return nx[0], ny[0], nz[0]

    v0 = jnp.where(q == 0, jnp.float32(1.0), jnp.float32(0.0))
    x00 = xv[pl.ds(0, _L)][0]
    y00 = yv[pl.ds(0, _L)][0]
    z00 = zv[pl.ds(0, _L)][0]
    cx0, cy0, cz0 = exchange(1, v0, jnp.float32(0.0), x00, y00, z00)

    def step(i, carry):
        cx, cy, cz = carry

        @pl.when(q == 0)
        def _():
            ivec = jnp.full((_L,), i, jnp.int32)
            plsc.store_scatter(cxv, [ivec], jnp.full((_L,), cx, jnp.float32),
                               mask=lane0)
            plsc.store_scatter(cyv, [ivec], jnp.full((_L,), cy, jnp.float32),
                               mask=lane0)
            plsc.store_scatter(czv, [ivec], jnp.full((_L,), cz, jnp.float32),
                               mask=lane0)

        def chunk(cc, car):
            mx, bi = car
            for k in range(8):
                off = cc * (8 * _L) + k * _L
                sl = pl.ds(off, _L)
                dx = xv[sl] - cx
                dy = yv[sl] - cy
                dz = zv[sl] - cz
                d = dx * dx + dy * dy
                d = d + dz * dz
                dn = jnp.minimum(dv[sl], d)
                dv[sl] = dn
                gt = dn > mx
                mx = jnp.where(gt, dn, mx)
                bi = jnp.where(gt, off + lanes, bi)
            return mx, bi

        mx, bi = lax.fori_loop(
            0, nq // (8 * _L), chunk,
            (jnp.full((_L,), -1.0, jnp.float32), jnp.zeros((_L,), jnp.int32)))
        m = jnp.max(mx)
        cand = jnp.where(mx == m, bi, nq)
        bl = jnp.min(cand)
        cvx = plsc.load_gather(xv, [jnp.full((_L,), bl, jnp.int32)])
        cvy = plsc.load_gather(yv, [jnp.full((_L,), bl, jnp.int32)])
        cvz = plsc.load_gather(zv, [jnp.full((_L,), bl, jnp.int32)])
        bgf = (qoff + bl).astype(jnp.float32)
        return exchange(i & 1, m, bgf, cvx, cvy, cvz)

    lax.fori_loop(0, _S, step, (cx0, cy0, cz0))

    @pl.when(q == 0)
    def _():
        pltpu.sync_copy(cxv, ocx.at[b])
        pltpu.sync_copy(cyv, ocy.at[b])
        pltpu.sync_copy(czv, ocz.at[b])


def _ball_body(x_hbm, y_hbm, z_hbm, cx_hbm, cy_hbm, cz_hbm, r_hbm, x0_hbm,
               xv, yv, zv, ccx, ccy, ccz, cntb, idxb, rows, sem):
    w = lax.axis_index("s") * 2 + lax.axis_index("c")
    b = w // 4
    q = w % 4
    sc = _S // 4
    pltpu.sync_copy(x_hbm.at[b], xv)
    pltpu.sync_copy(y_hbm.at[b], yv)
    pltpu.sync_copy(z_hbm.at[b], zv)
    pltpu.sync_copy(cx_hbm.at[b, pl.ds(q * sc, sc)], ccx)
    pltpu.sync_copy(cy_hbm.at[b, pl.ds(q * sc, sc)], ccy)
    pltpu.sync_copy(cz_hbm.at[b, pl.ds(q * sc, sc)], ccz)
    lanes = lax.iota(jnp.int32, _L)
    gb = b * _N
    gbv = jnp.full((_L,), gb, jnp.int32)

    def group(g, carry):
        slc = pl.ds(g * _L, _L)
        cxg = ccx[slc]
        cyg = ccy[slc]
        czg = ccz[slc]
        slotbase = (g * _L + lanes) * _K
        plsc.store_scatter(idxb, [slotbase], gbv)

        def pchunk(c, cnt):
            sl = pl.ds(c * _L, _L)
            xs = xv[sl]
            ys = yv[sl]
            zs = zv[sl]
            for k in range(_L):
                dx = cxg - xs[k]
                dy = cyg - ys[k]
                dz = czg - zs[k]
                d = dx * dx + dy * dy
                d = d + dz * dz
                msk = d < _R2
                okm = jnp.logical_and(msk, cnt < _K)
                gidx = jnp.full((_L,), gb + c * _L + k, jnp.int32)
                plsc.store_scatter(idxb, [slotbase + cnt], gidx, mask=okm)
                cnt = cnt + msk.astype(jnp.int32)
            return cnt

        cnt = lax.fori_loop(0, _N // _L, pchunk, jnp.zeros((_L,), jnp.int32))
        cntb[slc] = jnp.minimum(cnt, _K)
        return carry

    lax.fori_loop(0, sc // _L, group, 0)

    def block(blk, carry):
        for t in range(4):
            cl = blk * 4 + t
            clv = jnp.full((_L,), cl, jnp.int32)
            cntv = plsc.load_gather(cntb, [clv])
            firstv = plsc.load_gather(idxb, [clv * _K])
            for j in range(_K // _L):
                sl = pl.ds(cl * _K + j * _L, _L)
                keep = (j * _L + lanes) < cntv
                idxb[sl] = jnp.where(keep, idxb[sl], firstv)
        pltpu.async_copy(r_hbm.at[idxb.at[pl.ds(blk * 128, 128)]],
                         rows, sem).wait()
        for t in range(4):
            cl = blk * 4 + t
            clv = jnp.full((_L,), cl, jnp.int32)
            ctx = plsc.load_gather(ccx, [clv])
            cty = plsc.load_gather(ccy, [clv])
            ctz = plsc.load_gather(ccz, [clv])
            for coord, ctv in ((64, ctx), (65, cty), (66, ctz)):
                colv = jnp.full((_L,), coord, jnp.int32)
                for j in range(_K // _L):
                    ridx = t * _K + j * _L + lanes
                    vals = plsc.load_gather(rows, [ridx, colv])
                    plsc.store_scatter(rows, [ridx, colv], vals - ctv)
        gc = (b * _S + q * sc + blk * 4) * _K
        pltpu.sync_copy(rows, x0_hbm.at[pl.ds(gc, 128)])
        return carry

    lax.fori_loop(0, sc // 4, block, 0)


@functools.lru_cache(maxsize=None)
def _sc_kernels():
    mesh = plsc.VectorSubcoreMesh(core_axis_name="c", subcore_axis_name="s",
                                  num_cores=2, num_subcores=16)
    cparams = pltpu.CompilerParams(needs_layout_passes=False,
                                   use_tc_tiling_on_sc=False)
    fps = pl.kernel(
        _fps_body,
        out_type=[jax.ShapeDtypeStruct((_B, _S), jnp.float32)] * 3,
        mesh=mesh,
        compiler_params=cparams,
        scratch_types=[
            pltpu.VMEM((_N // 4,), jnp.float32),
            pltpu.VMEM((_N // 4,), jnp.float32),
            pltpu.VMEM((_N // 4,), jnp.float32),
            pltpu.VMEM((_N // 4,), jnp.float32),
            pltpu.VMEM((_S,), jnp.float32),
            pltpu.VMEM((_S,), jnp.float32),
            pltpu.VMEM((_S,), jnp.float32),
            pltpu.VMEM((_L,), jnp.float32),
            pltpu.VMEM((4, _L), jnp.float32),
            pltpu.VMEM_SHARED((2, _L, _L), jnp.float32),
        ],
    )
    ball = pl.kernel(
        _ball_body,
        out_type=jax.ShapeDtypeStruct((_M, _CW), jnp.float32),
        mesh=mesh,
        compiler_params=cparams,
        scratch_types=[
            pltpu.VMEM((_N,), jnp.float32),
            pltpu.VMEM((_N,), jnp.float32),
            pltpu.VMEM((_N,), jnp.float32),
            pltpu.VMEM((_S // 4,), jnp.float32),
            pltpu.VMEM((_S // 4,), jnp.float32),
            pltpu.VMEM((_S // 4,), jnp.float32),
            pltpu.VMEM((_S // 4,), jnp.int32),
            pltpu.VMEM((_S // 4 * _K,), jnp.int32),
            pltpu.VMEM((128, _CW), jnp.float32),
            pltpu.SemaphoreType.DMA,
        ],
    )
    return fps, ball


def _moments_body(x_ref, g_ref, s_ref):
    i = pl.program_id(0)

    @pl.when(i == 0)
    def _():
        g_ref[...] = jnp.zeros_like(g_ref)
        s_ref[...] = jnp.zeros_like(s_ref)

    x = x_ref[...]
    g_ref[...] += lax.dot_general(x, x, (((0,), (0,)), ((), ())),
                                  preferred_element_type=jnp.float32)
    s_ref[...] += jnp.sum(x, axis=0, keepdims=True)


def _moments(x):
    m, c = x.shape
    return pl.pallas_call(
        _moments_body,
        grid=(m // _TM,),
        in_specs=[pl.BlockSpec((_TM, c), lambda i: (i, 0))],
        out_specs=[pl.BlockSpec((c, c), lambda i: (0, 0)),
                   pl.BlockSpec((1, c), lambda i: (0, 0))],
        out_shape=[jax.ShapeDtypeStruct((c, c), jnp.float32),
                   jax.ShapeDtypeStruct((1, c), jnp.float32)],
    )(x)


def _layer_body(x_ref, w_ref, b_ref, y_ref, g_ref, s_ref):
    i = pl.program_id(0)

    @pl.when(i == 0)
    def _():
        g_ref[...] = jnp.zeros_like(g_ref)
        s_ref[...] = jnp.zeros_like(s_ref)

    y = lax.dot_general(x_ref[...], w_ref[...], (((1,), (0,)), ((), ())),
                        preferred_element_type=jnp.float32)
    y = jnp.maximum(y + b_ref[...], 0.0)
    y_ref[...] = y
    g_ref[...] += lax.dot_general(y, y, (((0,), (0,)), ((), ())),
                                  preferred_element_type=jnp.float32)
    s_ref[...] += jnp.sum(y, axis=0, keepdims=True)


def _layer(x, w, b):
    m, c = x.shape
    o = w.shape[1]
    return pl.pallas_call(
        _layer_body,
        grid=(m // _TM,),
        in_specs=[pl.BlockSpec((_TM, c), lambda i: (i, 0)),
                  pl.BlockSpec((c, o), lambda i: (0, 0)),
                  pl.BlockSpec((1, o), lambda i: (0, 0))],
        out_specs=[pl.BlockSpec((_TM, o), lambda i: (i, 0)),
                   pl.BlockSpec((o, o), lambda i: (0, 0)),
                   pl.BlockSpec((1, o), lambda i: (0, 0))],
        out_shape=[jax.ShapeDtypeStruct((m, o), jnp.float32),
                   jax.ShapeDtypeStruct((o, o), jnp.float32),
                   jax.ShapeDtypeStruct((1, o), jnp.float32)],
    )(x, w, b.reshape(1, o))


def _final_body(x_ref, w_ref, b_ref, o_ref):
    y = lax.dot_general(x_ref[...], w_ref[...], (((1,), (0,)), ((), ())),
                        preferred_element_type=jnp.float32)
    y = jnp.maximum(y + b_ref[...], 0.0)
    parts = [jnp.max(y[j * _K:(j + 1) * _K, :], axis=0, keepdims=True)
             for j in range(_TM // _K)]
    o_ref[...] = jnp.concatenate(parts, axis=0)


def _final(x, w, b):
    m, c = x.shape
    o = w.shape[1]
    return pl.pallas_call(
        _final_body,
        grid=(m // _TM,),
        in_specs=[pl.BlockSpec((_TM, c), lambda i: (i, 0)),
                  pl.BlockSpec((c, o), lambda i: (0, 0)),
                  pl.BlockSpec((1, o), lambda i: (0, 0))],
        out_specs=pl.BlockSpec((_TM // _K, o), lambda i: (i, 0)),
        out_shape=jax.ShapeDtypeStruct((m // _K, o), jnp.float32),
    )(x, w, b.reshape(1, o))


def _fold(G, S, W, b, g, be):
    mu = S[0] / _M
    muW = mu @ W
    mean_y = muW + b
    T = G @ W
    ey2 = jnp.sum(W * T, axis=0) / _M + 2.0 * b * muW + b * b
    var = ey2 - mean_y * mean_y
    scale = g / jnp.sqrt(var + 1e-5)
    return W * scale[None, :], (b - mean_y) * scale + be


def kernel(xyz, points, W0, b0, g0, be0, W1, b1, g1, be1, W2, b2, g2, be2):
    xb = xyz[:, 0, :]
    yb = xyz[:, 1, :]
    zb = xyz[:, 2, :]
    fps_call, ball_call = _sc_kernels()
    cx, cy, cz = fps_call(xb, yb, zb)
    new_xyz = jnp.stack([cx, cy, cz], axis=1)

    R = jnp.concatenate(
        [points.transpose(0, 2, 1), xyz.transpose(0, 2, 1),
         jnp.zeros((_B, _N, _CW - 67), jnp.float32)], axis=-1,
    ).reshape(_B * _N, _CW)
    x0 = ball_call(xb, yb, zb, cx, cy, cz, R)

    W0e = jnp.zeros((_CW, W0.shape[0]), jnp.float32)
    W0e = W0e.at[0:64, :].set(W0[:, 3:67].T)
    W0e = W0e.at[64:67, :].set(W0[:, 0:3].T)

    G0, S0 = _moments(x0)
    W0f, b0f = _fold(G0, S0, W0e, b0, g0, be0)
    x1, G1, S1 = _layer(x0, W0f, b0f)
    W1f, b1f = _fold(G1, S1, W1.T, b1, g1, be1)
    x2, G2, S2 = _layer(x1, W1f, b1f)
    W2f, b2f = _fold(G2, S2, W2.T, b2, g2, be2)
    feats = _final(x2, W2f, b2f)
    new_features = feats.reshape(_B, _S, W2.shape[0]).transpose(0, 2, 1)
    return new_xyz, new_features
